# baseline (speedup 1.0000x reference)
import functools

import jax
import jax.numpy as jnp
from jax import lax
from jax.experimental import pallas as pl
from jax.experimental.pallas import tpu as pltpu
from jax.experimental.pallas import tpu_sc as plsc

NN = 10000
NE = 160000
DXD = 128
DED = 16
DGD = 16
H = 64

NTILES = 32
CH = 128
EP = 163840
GR0 = 15360
GR1 = 5120
SCH = EP // NTILES // CH
GCH = 2 * EP // NTILES // CH
NP = 10016
ZR = NP // 16
OR_ = NN // 16

BE = 2048
GE = EP // BE
BN = 2000
GN = NN // BN


@functools.cache
def _sc_kernels():
    mesh = plsc.VectorSubcoreMesh(core_axis_name="c", subcore_axis_name="s")

    nbg = 8

    @functools.partial(
        pl.kernel,
        out_type=jax.ShapeDtypeStruct((2 * EP, H), jnp.float32),
        mesh=mesh,
        compiler_params=pltpu.CompilerParams(use_tc_tiling_on_sc=False),
        scratch_types=[
            pltpu.VMEM((GR0,), jnp.int32),
            pltpu.VMEM((nbg, CH, H), jnp.float32),
        ] + [pltpu.SemaphoreType.DMA] * (2 * nbg),
    )
    def sc_gather(table, idx, out, idx_v, rows_v, *sems):
        gsems, wsems = sems[:nbg], sems[nbg:]
        cid = lax.axis_index("c")
        sid = lax.axis_index("s")

        def run(base, ngroups):
            def gather_src(j):
                return table.at[idx_v.at[pl.ds(j * CH, CH)]]

            def out_dst(j):
                return out.at[pl.ds(base + j * CH, CH)]

            pltpu.sync_copy(idx.at[pl.ds(base, ngroups * nbg * CH)],
                            idx_v.at[pl.ds(0, ngroups * nbg * CH)])

            def body(g, carry):
                for b in range(nbg):
                    j = g * nbg + b

                    @pl.when(g > 0)
                    def _():
                        pltpu.make_async_copy(
                            rows_v.at[b], out_dst(j - nbg), wsems[b]).wait()

                    pltpu.async_copy(gather_src(j), rows_v.at[b], gsems[b])
                for b in range(nbg):
                    j = g * nbg + b
                    pltpu.make_async_copy(gather_src(j), rows_v.at[b],
                                          gsems[b]).wait()
                    pltpu.async_copy(rows_v.at[b], out_dst(j), wsems[b])
                return carry

            lax.fori_loop(0, ngroups, body, 0)
            for b in range(nbg):
                j = (ngroups - 1) * nbg + b
                pltpu.make_async_copy(rows_v.at[b], out_dst(j), wsems[b]).wait()

        @pl.when(cid == 0)
        def _():
            run(sid * GR0, GR0 // CH // nbg)

        @pl.when(cid == 1)
        def _():
            run(16 * GR0 + sid * GR1, GR1 // CH // nbg)

    nbs = 4

    @functools.partial(
        pl.kernel,
        out_type=jax.ShapeDtypeStruct((2, NN, H), jnp.float32),
        mesh=mesh,
        compiler_params=pltpu.CompilerParams(use_tc_tiling_on_sc=False),
        scratch_types=[
            pltpu.VMEM_SHARED((NP, H), jnp.float32),
            pltpu.VMEM((SCH, CH), jnp.int32),
            pltpu.VMEM((nbs, CH, H), jnp.float32),
        ] + [pltpu.SemaphoreType.DMA] * (2 * nbs),
    )
    def sc_scatter(e2, idx3, zeros_hbm, out, shared, idx_v, rows_v, *sems):
        rsems, ssems = sems[:nbs], sems[nbs:]
        cid = lax.axis_index("c")
        sid = lax.axis_index("s")
        wid = sid * 2 + cid
        pltpu.sync_copy(zeros_hbm.at[pl.ds(sid * ZR, ZR)],
                        shared.at[pl.ds(sid * ZR, ZR)])
        pltpu.sync_copy(idx3.at[wid], idx_v)
        plsc.subcore_barrier()
        base = wid * (EP // NTILES)

        def body(g, carry):
            for b in range(nbs):
                j = g * nbs + b

                @pl.when(g > 0)
                def _():
                    pltpu.make_async_copy(
                        rows_v.at[b], shared.at[idx_v.at[j - nbs]], ssems[b]).wait()

                pltpu.async_copy(e2.at[pl.ds(base + j * CH, CH)],
                                 rows_v.at[b], rsems[b])
            for b in range(nbs):
                j = g * nbs + b
                pltpu.make_async_copy(e2.at[pl.ds(base + j * CH, CH)],
                                      rows_v.at[b], rsems[b]).wait()
                pltpu.async_copy(rows_v.at[b], shared.at[idx_v.at[j]],
                                 ssems[b], add=True)
            return carry

        ngroups = SCH // nbs
        lax.fori_loop(0, ngroups, body, 0)
        for b in range(nbs):
            j = (ngroups - 1) * nbs + b
            pltpu.make_async_copy(rows_v.at[b], shared.at[idx_v.at[j]],
                                  ssems[b]).wait()
        plsc.subcore_barrier()
        pltpu.sync_copy(shared.at[pl.ds(sid * OR_, OR_)],
                        out.at[cid, pl.ds(sid * OR_, OR_)])

    return sc_gather, sc_scatter


def _sc_gather(table, idx):
    return _sc_kernels()[0](table, idx)


def _sc_scatter(e2, idx3, zeros_np):
    return _sc_kernels()[1](e2, idx3, zeros_np)


def _ln(h):
    m = jnp.mean(h, axis=-1, keepdims=True)
    v = jnp.var(h, axis=-1, keepdims=True)
    return (h - m) / jnp.sqrt(v + 1e-5)


def _dot(a, b):
    return jax.lax.dot_general(a, b, (((1,), (0,)), ((), ())),
                               preferred_element_type=jnp.float32)


_W64 = pl.BlockSpec((H, H), lambda i: (0, 0))
_B64 = pl.BlockSpec((1, H), lambda i: (0, 0))


def _edge_body(has_de, last, *refs):
    if has_de:
        (e_ref, de_ref, gsd_ref, we, be, w1a, w1b, b1, w2, b2, dw, db, ow, ob,
         e2_ref, de2_ref, sum_ref, *oe) = refs
    else:
        (e_ref, gsd_ref, we, be, w1a, w1b, b1, w2, b2, dw, db, ow, ob,
         e2_ref, de2_ref, sum_ref, *oe) = refs
        de_ref = None
    i = pl.program_id(0)
    le = jnp.maximum(_dot(e_ref[...], we[...]) + be[...], 0.0)
    de = de_ref[...] if has_de else le
    h = _dot(le, w1a[...]) + _dot(de, w1b[...])
    h = h + gsd_ref[0] + gsd_ref[1] + b1[...]
    h = jnp.maximum(h, 0.0)
    h = jnp.maximum(_dot(h, w2[...]) + b2[...], 0.0)
    e2 = _ln(h)
    e2_ref[...] = e2
    de2 = jnp.maximum(_dot(e2, dw[...]) + db[...], 0.0)
    de2_ref[...] = de2
    rows = i * BE + lax.broadcasted_iota(jnp.int32, (BE, 1), 0)
    part = jnp.sum(jnp.where(rows < NE, e2, 0.0), axis=0, keepdims=True)

    @pl.when(i == 0)
    def _():
        sum_ref[...] = jnp.zeros_like(sum_ref)

    sum_ref[...] += part
    if last:
        oe[0][...] = _dot(de2, ow[...]) + ob[...]


def _make_edge(has_de, last):
    in_specs = [pl.BlockSpec((BE, DED), lambda i: (i, 0))]
    if has_de:
        in_specs.append(pl.BlockSpec((BE, H), lambda i: (i, 0)))
    in_specs.append(pl.BlockSpec((2, BE, H), lambda i: (0, i, 0)))
    in_specs += [pl.BlockSpec((DED, H), lambda i: (0, 0)), _B64,
                 _W64, _W64, _B64, _W64, _B64, _W64, _B64,
                 pl.BlockSpec((H, DED), lambda i: (0, 0)),
                 pl.BlockSpec((1, DED), lambda i: (0, 0))]
    out_shape = [jax.ShapeDtypeStruct((EP, H), jnp.float32),
                 jax.ShapeDtypeStruct((EP, H), jnp.float32),
                 jax.ShapeDtypeStruct((1, H), jnp.float32)]
    out_specs = [pl.BlockSpec((BE, H), lambda i: (i, 0)),
                 pl.BlockSpec((BE, H), lambda i: (i, 0)),
                 pl.BlockSpec((1, H), lambda i: (0, 0))]
    if last:
        out_shape.append(jax.ShapeDtypeStruct((EP, DED), jnp.float32))
        out_specs.append(pl.BlockSpec((BE, DED), lambda i: (i, 0)))
    return pl.pallas_call(
        functools.partial(_edge_body, has_de, last),
        grid=(GE,), in_specs=in_specs, out_specs=out_specs, out_shape=out_shape)


_edge0 = _make_edge(False, False)
_edge1 = _make_edge(True, True)


def _node_body(has_dx, last, *refs):
    if has_dx:
        (lx_ref, dx_ref, agg_ref, wn1a, wn1b, wn1c, bn1, wn2, bn2, dw, db,
         wsa, wsb, wda, wdb, ow, ob, dx2_ref, xsd_ref, sum_ref, *ox) = refs
    else:
        (lx_ref, agg_ref, wn1a, wn1b, wn1c, bn1, wn2, bn2, dw, db,
         wsa, wsb, wda, wdb, ow, ob, dx2_ref, xsd_ref, sum_ref, *ox) = refs
        dx_ref = lx_ref
    i = pl.program_id(0)
    lx = lx_ref[...]
    agg = agg_ref[0] + agg_ref[1]
    h = _dot(lx, wn1a[...]) + _dot(dx_ref[...], wn1b[...]) + _dot(agg, wn1c[...]) + bn1[...]
    h = jnp.maximum(h, 0.0)
    h = jnp.maximum(_dot(h, wn2[...]) + bn2[...], 0.0)
    x2 = _ln(h)
    dx2 = jnp.maximum(_dot(x2, dw[...]) + db[...], 0.0)
    dx2_ref[...] = dx2
    xsd_ref[0] = _dot(lx, wsa[...]) + _dot(dx2, wsb[...])
    xsd_ref[1] = _dot(lx, wda[...]) + _dot(dx2, wdb[...])
    part = jnp.sum(x2, axis=0, keepdims=True)

    @pl.when(i == 0)
    def _():
        sum_ref[...] = jnp.zeros_like(sum_ref)

    sum_ref[...] += part
    if last:
        ox[0][...] = _dot(dx2, ow[...]) + ob[...]


def _make_node(has_dx, last):
    in_specs = [pl.BlockSpec((BN, H), lambda i: (i, 0))]
    if has_dx:
        in_specs.append(pl.BlockSpec((BN, H), lambda i: (i, 0)))
    in_specs.append(pl.BlockSpec((2, BN, H), lambda i: (0, i, 0)))
    in_specs += [_W64, _W64, _W64, _B64, _W64, _B64, _W64, _B64,
                 _W64, _W64, _W64, _W64,
                 pl.BlockSpec((H, DXD), lambda i: (0, 0)),
                 pl.BlockSpec((1, DXD), lambda i: (0, 0))]
    out_shape = [jax.ShapeDtypeStruct((NN, H), jnp.float32),
                 jax.ShapeDtypeStruct((2, NN, H), jnp.float32),
                 jax.ShapeDtypeStruct((1, H), jnp.float32)]
    out_specs = [pl.BlockSpec((BN, H), lambda i: (i, 0)),
                 pl.BlockSpec((2, BN, H), lambda i: (0, i, 0)),
                 pl.BlockSpec((1, H), lambda i: (0, 0))]
    if last:
        out_shape.append(jax.ShapeDtypeStruct((NN, DXD), jnp.float32))
        out_specs.append(pl.BlockSpec((BN, DXD), lambda i: (i, 0)))
    return pl.pallas_call(
        functools.partial(_node_body, has_dx, last),
        grid=(GN,), in_specs=in_specs, out_specs=out_specs, out_shape=out_shape)


_node0 = _make_node(False, False)
_node1 = _make_node(True, True)


def _enc_e_body(e_ref, w_ref, b_ref, le_ref):
    le_ref[...] = jnp.maximum(_dot(e_ref[...], w_ref[...]) + b_ref[...], 0.0)


_enc_e = pl.pallas_call(
    _enc_e_body, grid=(GE,),
    in_specs=[pl.BlockSpec((BE, DED), lambda i: (i, 0)),
              pl.BlockSpec((DED, H), lambda i: (0, 0)),
              pl.BlockSpec((1, H), lambda i: (0, 0))],
    out_specs=pl.BlockSpec((BE, H), lambda i: (i, 0)),
    out_shape=jax.ShapeDtypeStruct((EP, H), jnp.float32))


def _enc_x_body(x_ref, w_ref, b_ref, ws_ref, wd_ref, lx_ref, xsd_ref):
    lx = jnp.maximum(_dot(x_ref[...], w_ref[...]) + b_ref[...], 0.0)
    lx_ref[...] = lx
    lxc = jnp.concatenate([lx, lx], axis=1)
    xsd_ref[0] = _dot(lxc, ws_ref[...])
    xsd_ref[1] = _dot(lxc, wd_ref[...])


_enc_x = pl.pallas_call(
    _enc_x_body, grid=(GN,),
    in_specs=[pl.BlockSpec((BN, DXD), lambda i: (i, 0)),
              pl.BlockSpec((DXD, H), lambda i: (0, 0)),
              pl.BlockSpec((1, H), lambda i: (0, 0)),
              pl.BlockSpec((2 * H, H), lambda i: (0, 0)),
              pl.BlockSpec((2 * H, H), lambda i: (0, 0))],
    out_specs=[pl.BlockSpec((BN, H), lambda i: (i, 0)),
               pl.BlockSpec((2, BN, H), lambda i: (0, i, 0))],
    out_shape=[jax.ShapeDtypeStruct((NN, H), jnp.float32),
               jax.ShapeDtypeStruct((2, NN, H), jnp.float32)])


def _prep_g_body(g_ref, w_ref, b_ref, wge_ref, b1e_ref, wgn_ref, b1n_ref,
                 lg_ref, be_ref, bn_ref):
    lg = jnp.maximum(_dot(g_ref[...], w_ref[...]) + b_ref[...], 0.0)
    lg_ref[...] = lg
    lgc = jnp.concatenate([lg, lg], axis=1)
    be_ref[...] = _dot(lgc, wge_ref[...]) + b1e_ref[...]
    bn_ref[...] = _dot(lgc, wgn_ref[...]) + b1n_ref[...]


_prep_g = pl.pallas_call(
    _prep_g_body,
    out_shape=[jax.ShapeDtypeStruct((1, H), jnp.float32)] * 3)


def _glob_body(last, *refs):
    (lg_ref, dg_ref, se_ref, sx_ref, wg1, bg1, wg2, bg2, dw, db,
     wge, b1e, wgn, b1n, ow, ob, *outs) = refs
    gcat = jnp.concatenate([lg_ref[...], dg_ref[...]], axis=1)
    gin = jnp.concatenate([gcat, se_ref[...], sx_ref[...]], axis=1)
    h = jnp.maximum(_dot(gin, wg1[...]) + bg1[...], 0.0)
    h = jnp.maximum(_dot(h, wg2[...]) + bg2[...], 0.0)
    g2 = _ln(h)
    dg2 = jnp.maximum(_dot(g2, dw[...]) + db[...], 0.0)
    if last:
        outs[0][...] = _dot(dg2, ow[...]) + ob[...]
    else:
        dg_out, be_out, bn_out = outs
        dg_out[...] = dg2
        gcat2 = jnp.concatenate([lg_ref[...], dg2], axis=1)
        be_out[...] = _dot(gcat2, wge[...]) + b1e[...]
        bn_out[...] = _dot(gcat2, wgn[...]) + b1n[...]


_glob0 = pl.pallas_call(
    functools.partial(_glob_body, False),
    out_shape=[jax.ShapeDtypeStruct((1, H), jnp.float32)] * 3)
_glob1 = pl.pallas_call(
    functools.partial(_glob_body, True),
    out_shape=[jax.ShapeDtypeStruct((1, DGD), jnp.float32)])


def _row(b):
    return b[None, :]


def kernel(x, e, g, params, edges, node_idx, edge_idx, steps):
    p = params
    src = edges[0]
    dst = edges[1]
    padn = EP - NE
    zpad = jnp.zeros((padn,), jnp.int32)
    idx_gather = jnp.concatenate([src, zpad, dst + NN, zpad,
                                  jnp.zeros((GR0 - GR1,), jnp.int32)])
    idx_scatter = jnp.concatenate(
        [dst, jnp.full((padn,), NN, jnp.int32)]).reshape(NTILES, SCH, CH)
    zeros_np = jnp.zeros((NP, H), jnp.float32)
    e_pad = jnp.pad(e, ((0, padn), (0, 0)))

    w1 = p['core_e_W1']
    wn1 = p['core_n_W1']

    lx, xsd = _enc_x(x, p['enc_x_W'], _row(p['enc_x_b']),
                     w1[128:256], w1[256:384])
    lg, be_b, bn_b = _prep_g(g, p['enc_g_W'], _row(p['enc_g_b']),
                             w1[384:512], _row(p['core_e_b1']),
                             wn1[192:320], _row(p['core_n_b1']))

    enc_e = (p['enc_e_W'], _row(p['enc_e_b']))
    edge_w = (w1[0:64], w1[64:128])
    edge_tail = (p['core_e_W2'], _row(p['core_e_b2']),
                 p['dec_e_W'], _row(p['dec_e_b']),
                 p['out_e_W'], _row(p['out_e_b']))
    node_w = (wn1[0:64], wn1[64:128], wn1[128:192])
    node_tail = (p['core_n_W2'], _row(p['core_n_b2']),
                 p['dec_x_W'], _row(p['dec_x_b']),
                 w1[128:192], w1[192:256], w1[256:320], w1[320:384],
                 p['out_x_W'], _row(p['out_x_b']))
    glob_w = (p['core_g_W1'], _row(p['core_g_b1']),
              p['core_g_W2'], _row(p['core_g_b2']),
              p['dec_g_W'], _row(p['dec_g_b']),
              w1[384:512], _row(p['core_e_b1']),
              wn1[192:320], _row(p['core_n_b1']),
              p['out_g_W'], _row(p['out_g_b']))

    gsd = _sc_gather(xsd.reshape(2 * NN, H), idx_gather).reshape(2, EP, H)
    e2, de, se = _edge0(e_pad, gsd, *enc_e, edge_w[0],
                        edge_w[1], be_b, *edge_tail)
    agg = _sc_scatter(e2, idx_scatter, zeros_np)
    dx, xsd, sx = _node0(lx, agg, *node_w, bn_b, *node_tail)
    dg, be_b, bn_b = _glob0(lg, lg, se, sx, *glob_w)

    gsd = _sc_gather(xsd.reshape(2 * NN, H), idx_gather).reshape(2, EP, H)
    e2, de, se, oe = _edge1(e_pad, de, gsd, *enc_e, edge_w[0],
                            edge_w[1], be_b, *edge_tail)
    agg = _sc_scatter(e2, idx_scatter, zeros_np)
    dx, _, sx, ox = _node1(lx, dx, agg, *node_w, bn_b, *node_tail)
    (og,) = _glob1(lg, dg, se, sx, *glob_w)

    return (oe[:NE], ox, og)


# speedup vs baseline: 1.1156x; 1.1156x over previous
import functools

import jax
import jax.numpy as jnp
from jax import lax
from jax.experimental import pallas as pl
from jax.experimental.pallas import tpu as pltpu
from jax.experimental.pallas import tpu_sc as plsc

NN = 10000
NE = 160000
DXD = 128
DED = 16
DGD = 16
H = 64

NTILES = 32
CH = 128
EP = 163840
GR0 = 15360
GR1 = 5120
SCH = EP // NTILES // CH
GCH = 2 * EP // NTILES // CH
NP = 10016
ZR = NP // 16
OR_ = NN // 16

BE = 2048
GE = EP // BE
BN = 2000
GN = NN // BN


@functools.cache
def _sc_kernels():
    mesh = plsc.VectorSubcoreMesh(core_axis_name="c", subcore_axis_name="s")

    nbg = 8

    @functools.partial(
        pl.kernel,
        out_type=jax.ShapeDtypeStruct((2 * EP, H), jnp.float32),
        mesh=mesh,
        compiler_params=pltpu.CompilerParams(use_tc_tiling_on_sc=False),
        scratch_types=[
            pltpu.VMEM((GR0,), jnp.int32),
            pltpu.VMEM((nbg, CH, H), jnp.float32),
        ] + [pltpu.SemaphoreType.DMA] * (2 * nbg),
    )
    def sc_gather(table, idx, out, idx_v, rows_v, *sems):
        gsems, wsems = sems[:nbg], sems[nbg:]
        cid = lax.axis_index("c")
        sid = lax.axis_index("s")

        def run(base, ngroups):
            def gather_src(j):
                return table.at[idx_v.at[pl.ds(j * CH, CH)]]

            def out_dst(j):
                return out.at[pl.ds(base + j * CH, CH)]

            pltpu.sync_copy(idx.at[pl.ds(base, ngroups * nbg * CH)],
                            idx_v.at[pl.ds(0, ngroups * nbg * CH)])

            def body(g, carry):
                for b in range(nbg):
                    j = g * nbg + b

                    @pl.when(g > 0)
                    def _():
                        pltpu.make_async_copy(
                            rows_v.at[b], out_dst(j - nbg), wsems[b]).wait()

                    pltpu.async_copy(gather_src(j), rows_v.at[b], gsems[b])
                for b in range(nbg):
                    j = g * nbg + b
                    pltpu.make_async_copy(gather_src(j), rows_v.at[b],
                                          gsems[b]).wait()
                    pltpu.async_copy(rows_v.at[b], out_dst(j), wsems[b])
                return carry

            lax.fori_loop(0, ngroups, body, 0)
            for b in range(nbg):
                j = (ngroups - 1) * nbg + b
                pltpu.make_async_copy(rows_v.at[b], out_dst(j), wsems[b]).wait()

        @pl.when(cid == 0)
        def _():
            run(sid * GR0, GR0 // CH // nbg)

        @pl.when(cid == 1)
        def _():
            run(16 * GR0 + sid * GR1, GR1 // CH // nbg)

    nbs = 4

    @functools.partial(
        pl.kernel,
        out_type=jax.ShapeDtypeStruct((2, NN, H), jnp.float32),
        mesh=mesh,
        compiler_params=pltpu.CompilerParams(use_tc_tiling_on_sc=False),
        scratch_types=[
            pltpu.VMEM_SHARED((NP, H), jnp.float32),
            pltpu.VMEM((SCH, CH), jnp.int32),
            pltpu.VMEM((nbs, CH, H), jnp.float32),
        ] + [pltpu.SemaphoreType.DMA] * (2 * nbs),
    )
    def sc_scatter(e2, idx3, zeros_hbm, out, shared, idx_v, rows_v, *sems):
        rsems, ssems = sems[:nbs], sems[nbs:]
        cid = lax.axis_index("c")
        sid = lax.axis_index("s")
        wid = sid * 2 + cid
        pltpu.sync_copy(zeros_hbm.at[pl.ds(sid * ZR, ZR)],
                        shared.at[pl.ds(sid * ZR, ZR)])
        pltpu.sync_copy(idx3.at[wid], idx_v)
        plsc.subcore_barrier()
        base = wid * (EP // NTILES)

        def body(g, carry):
            for b in range(nbs):
                j = g * nbs + b

                @pl.when(g > 0)
                def _():
                    pltpu.make_async_copy(
                        rows_v.at[b], shared.at[idx_v.at[j - nbs]], ssems[b]).wait()

                pltpu.async_copy(e2.at[pl.ds(base + j * CH, CH)],
                                 rows_v.at[b], rsems[b])
            for b in range(nbs):
                j = g * nbs + b
                pltpu.make_async_copy(e2.at[pl.ds(base + j * CH, CH)],
                                      rows_v.at[b], rsems[b]).wait()
                pltpu.async_copy(rows_v.at[b], shared.at[idx_v.at[j]],
                                 ssems[b], add=True)
            return carry

        ngroups = SCH // nbs
        lax.fori_loop(0, ngroups, body, 0)
        for b in range(nbs):
            j = (ngroups - 1) * nbs + b
            pltpu.make_async_copy(rows_v.at[b], shared.at[idx_v.at[j]],
                                  ssems[b]).wait()
        plsc.subcore_barrier()
        pltpu.sync_copy(shared.at[pl.ds(sid * OR_, OR_)],
                        out.at[cid, pl.ds(sid * OR_, OR_)])

    return sc_gather, sc_scatter


def _sc_gather(table, idx):
    return _sc_kernels()[0](table, idx)


def _sc_scatter(e2, idx3, zeros_np):
    return _sc_kernels()[1](e2, idx3, zeros_np)


def _ln(h):
    m = jnp.mean(h, axis=-1, keepdims=True)
    v = jnp.var(h, axis=-1, keepdims=True)
    return (h - m) / jnp.sqrt(v + 1e-5)


def _dot(a, b):
    return jax.lax.dot_general(a, b, (((1,), (0,)), ((), ())),
                               preferred_element_type=jnp.float32)


_W64 = pl.BlockSpec((H, H), lambda i: (0, 0))
_B64 = pl.BlockSpec((1, H), lambda i: (0, 0))


def _edge_body(has_de, last, *refs):
    if has_de:
        (e_ref, de_ref, gsd_ref, we, be, w1a, w1b, b1, w2, b2, dw, db, ow, ob,
         e2_ref, de2_ref, sum_ref, *oe) = refs
    else:
        (e_ref, gsd_ref, we, be, w1a, w1b, b1, w2, b2, dw, db, ow, ob,
         e2_ref, de2_ref, sum_ref, *oe) = refs
        de_ref = None
    i = pl.program_id(0)
    le = jnp.maximum(_dot(e_ref[...], we[...]) + be[...], 0.0)
    de = de_ref[...] if has_de else le
    g = gsd_ref[...]
    h = _dot(le, w1a[...]) + _dot(de, w1b[...])
    h = h + g[:, :H] + g[:, H:] + b1[...]
    h = jnp.maximum(h, 0.0)
    h = jnp.maximum(_dot(h, w2[...]) + b2[...], 0.0)
    e2 = _ln(h)
    e2_ref[...] = e2
    de2 = jnp.maximum(_dot(e2, dw[...]) + db[...], 0.0)
    de2_ref[...] = de2
    rows = i * BE + lax.broadcasted_iota(jnp.int32, (BE, 1), 0)
    part = jnp.sum(jnp.where(rows < NE, e2, 0.0), axis=0, keepdims=True)

    @pl.when(i == 0)
    def _():
        sum_ref[...] = jnp.zeros_like(sum_ref)

    sum_ref[...] += part
    if last:
        oe[0][...] = _dot(de2, ow[...]) + ob[...]


def _make_edge(has_de, last):
    in_specs = [pl.BlockSpec((BE, DED), lambda i: (i, 0))]
    if has_de:
        in_specs.append(pl.BlockSpec((BE, H), lambda i: (i, 0)))
    in_specs.append(pl.BlockSpec((BE, 2 * H), lambda i: (i, 0)))
    in_specs += [pl.BlockSpec((DED, H), lambda i: (0, 0)), _B64,
                 _W64, _W64, _B64, _W64, _B64, _W64, _B64,
                 pl.BlockSpec((H, DED), lambda i: (0, 0)),
                 pl.BlockSpec((1, DED), lambda i: (0, 0))]
    out_shape = [jax.ShapeDtypeStruct((EP, H), jnp.float32),
                 jax.ShapeDtypeStruct((EP, H), jnp.float32),
                 jax.ShapeDtypeStruct((1, H), jnp.float32)]
    out_specs = [pl.BlockSpec((BE, H), lambda i: (i, 0)),
                 pl.BlockSpec((BE, H), lambda i: (i, 0)),
                 pl.BlockSpec((1, H), lambda i: (0, 0))]
    if last:
        out_shape.append(jax.ShapeDtypeStruct((EP, DED), jnp.float32))
        out_specs.append(pl.BlockSpec((BE, DED), lambda i: (i, 0)))
    return pl.pallas_call(
        functools.partial(_edge_body, has_de, last),
        grid=(GE,), in_specs=in_specs, out_specs=out_specs, out_shape=out_shape)


_edge0 = _make_edge(False, False)
_edge1 = _make_edge(True, True)


def _node_body(has_dx, last, *refs):
    if has_dx:
        (lx_ref, dx_ref, agg_ref, wn1a, wn1b, wn1c, bn1, wn2, bn2, dw, db,
         wsa, wsb, wda, wdb, ow, ob, dx2_ref, xsd_ref, sum_ref, *ox) = refs
    else:
        (lx_ref, agg_ref, wn1a, wn1b, wn1c, bn1, wn2, bn2, dw, db,
         wsa, wsb, wda, wdb, ow, ob, dx2_ref, xsd_ref, sum_ref, *ox) = refs
        dx_ref = lx_ref
    i = pl.program_id(0)
    lx = lx_ref[...]
    agg = agg_ref[0] + agg_ref[1]
    h = _dot(lx, wn1a[...]) + _dot(dx_ref[...], wn1b[...]) + _dot(agg, wn1c[...]) + bn1[...]
    h = jnp.maximum(h, 0.0)
    h = jnp.maximum(_dot(h, wn2[...]) + bn2[...], 0.0)
    x2 = _ln(h)
    dx2 = jnp.maximum(_dot(x2, dw[...]) + db[...], 0.0)
    dx2_ref[...] = dx2
    xsd_ref[0] = _dot(lx, wsa[...]) + _dot(dx2, wsb[...])
    xsd_ref[1] = _dot(lx, wda[...]) + _dot(dx2, wdb[...])
    part = jnp.sum(x2, axis=0, keepdims=True)

    @pl.when(i == 0)
    def _():
        sum_ref[...] = jnp.zeros_like(sum_ref)

    sum_ref[...] += part
    if last:
        ox[0][...] = _dot(dx2, ow[...]) + ob[...]


def _make_node(has_dx, last):
    in_specs = [pl.BlockSpec((BN, H), lambda i: (i, 0))]
    if has_dx:
        in_specs.append(pl.BlockSpec((BN, H), lambda i: (i, 0)))
    in_specs.append(pl.BlockSpec((2, BN, H), lambda i: (0, i, 0)))
    in_specs += [_W64, _W64, _W64, _B64, _W64, _B64, _W64, _B64,
                 _W64, _W64, _W64, _W64,
                 pl.BlockSpec((H, DXD), lambda i: (0, 0)),
                 pl.BlockSpec((1, DXD), lambda i: (0, 0))]
    out_shape = [jax.ShapeDtypeStruct((NN, H), jnp.float32),
                 jax.ShapeDtypeStruct((2, NN, H), jnp.float32),
                 jax.ShapeDtypeStruct((1, H), jnp.float32)]
    out_specs = [pl.BlockSpec((BN, H), lambda i: (i, 0)),
                 pl.BlockSpec((2, BN, H), lambda i: (0, i, 0)),
                 pl.BlockSpec((1, H), lambda i: (0, 0))]
    if last:
        out_shape.append(jax.ShapeDtypeStruct((NN, DXD), jnp.float32))
        out_specs.append(pl.BlockSpec((BN, DXD), lambda i: (i, 0)))
    return pl.pallas_call(
        functools.partial(_node_body, has_dx, last),
        grid=(GN,), in_specs=in_specs, out_specs=out_specs, out_shape=out_shape)


_node0 = _make_node(False, False)
_node1 = _make_node(True, True)


def _enc_e_body(e_ref, w_ref, b_ref, le_ref):
    le_ref[...] = jnp.maximum(_dot(e_ref[...], w_ref[...]) + b_ref[...], 0.0)


_enc_e = pl.pallas_call(
    _enc_e_body, grid=(GE,),
    in_specs=[pl.BlockSpec((BE, DED), lambda i: (i, 0)),
              pl.BlockSpec((DED, H), lambda i: (0, 0)),
              pl.BlockSpec((1, H), lambda i: (0, 0))],
    out_specs=pl.BlockSpec((BE, H), lambda i: (i, 0)),
    out_shape=jax.ShapeDtypeStruct((EP, H), jnp.float32))


def _enc_x_body(x_ref, w_ref, b_ref, ws_ref, wd_ref, lx_ref, xsd_ref):
    lx = jnp.maximum(_dot(x_ref[...], w_ref[...]) + b_ref[...], 0.0)
    lx_ref[...] = lx
    lxc = jnp.concatenate([lx, lx], axis=1)
    xsd_ref[0] = _dot(lxc, ws_ref[...])
    xsd_ref[1] = _dot(lxc, wd_ref[...])


_enc_x = pl.pallas_call(
    _enc_x_body, grid=(GN,),
    in_specs=[pl.BlockSpec((BN, DXD), lambda i: (i, 0)),
              pl.BlockSpec((DXD, H), lambda i: (0, 0)),
              pl.BlockSpec((1, H), lambda i: (0, 0)),
              pl.BlockSpec((2 * H, H), lambda i: (0, 0)),
              pl.BlockSpec((2 * H, H), lambda i: (0, 0))],
    out_specs=[pl.BlockSpec((BN, H), lambda i: (i, 0)),
               pl.BlockSpec((2, BN, H), lambda i: (0, i, 0))],
    out_shape=[jax.ShapeDtypeStruct((NN, H), jnp.float32),
               jax.ShapeDtypeStruct((2, NN, H), jnp.float32)])


def _prep_g_body(g_ref, w_ref, b_ref, wge_ref, b1e_ref, wgn_ref, b1n_ref,
                 lg_ref, be_ref, bn_ref):
    lg = jnp.maximum(_dot(g_ref[...], w_ref[...]) + b_ref[...], 0.0)
    lg_ref[...] = lg
    lgc = jnp.concatenate([lg, lg], axis=1)
    be_ref[...] = _dot(lgc, wge_ref[...]) + b1e_ref[...]
    bn_ref[...] = _dot(lgc, wgn_ref[...]) + b1n_ref[...]


_prep_g = pl.pallas_call(
    _prep_g_body,
    out_shape=[jax.ShapeDtypeStruct((1, H), jnp.float32)] * 3)


def _glob_body(last, *refs):
    (lg_ref, dg_ref, se_ref, sx_ref, wg1, bg1, wg2, bg2, dw, db,
     wge, b1e, wgn, b1n, ow, ob, *outs) = refs
    gcat = jnp.concatenate([lg_ref[...], dg_ref[...]], axis=1)
    gin = jnp.concatenate([gcat, se_ref[...], sx_ref[...]], axis=1)
    h = jnp.maximum(_dot(gin, wg1[...]) + bg1[...], 0.0)
    h = jnp.maximum(_dot(h, wg2[...]) + bg2[...], 0.0)
    g2 = _ln(h)
    dg2 = jnp.maximum(_dot(g2, dw[...]) + db[...], 0.0)
    if last:
        outs[0][...] = _dot(dg2, ow[...]) + ob[...]
    else:
        dg_out, be_out, bn_out = outs
        dg_out[...] = dg2
        gcat2 = jnp.concatenate([lg_ref[...], dg2], axis=1)
        be_out[...] = _dot(gcat2, wge[...]) + b1e[...]
        bn_out[...] = _dot(gcat2, wgn[...]) + b1n[...]


_glob0 = pl.pallas_call(
    functools.partial(_glob_body, False),
    out_shape=[jax.ShapeDtypeStruct((1, H), jnp.float32)] * 3)
_glob1 = pl.pallas_call(
    functools.partial(_glob_body, True),
    out_shape=[jax.ShapeDtypeStruct((1, DGD), jnp.float32)])


def _row(b):
    return b[None, :]


def kernel(x, e, g, params, edges, node_idx, edge_idx, steps):
    p = params
    src = edges[0]
    dst = edges[1]
    padn = EP - NE
    zpad = jnp.zeros((padn,), jnp.int32)
    idx_gather = jnp.stack(
        [jnp.concatenate([src, zpad]),
         jnp.concatenate([dst + NN, zpad])], axis=1).reshape(-1)
    idx_scatter = jnp.concatenate(
        [dst, jnp.full((padn,), NN, jnp.int32)]).reshape(NTILES, SCH, CH)
    zeros_np = jnp.zeros((NP, H), jnp.float32)
    e_pad = jnp.pad(e, ((0, padn), (0, 0)))

    w1 = p['core_e_W1']
    wn1 = p['core_n_W1']

    lx, xsd = _enc_x(x, p['enc_x_W'], _row(p['enc_x_b']),
                     w1[128:256], w1[256:384])
    lg, be_b, bn_b = _prep_g(g, p['enc_g_W'], _row(p['enc_g_b']),
                             w1[384:512], _row(p['core_e_b1']),
                             wn1[192:320], _row(p['core_n_b1']))

    enc_e = (p['enc_e_W'], _row(p['enc_e_b']))
    edge_w = (w1[0:64], w1[64:128])
    edge_tail = (p['core_e_W2'], _row(p['core_e_b2']),
                 p['dec_e_W'], _row(p['dec_e_b']),
                 p['out_e_W'], _row(p['out_e_b']))
    node_w = (wn1[0:64], wn1[64:128], wn1[128:192])
    node_tail = (p['core_n_W2'], _row(p['core_n_b2']),
                 p['dec_x_W'], _row(p['dec_x_b']),
                 w1[128:192], w1[192:256], w1[256:320], w1[320:384],
                 p['out_x_W'], _row(p['out_x_b']))
    glob_w = (p['core_g_W1'], _row(p['core_g_b1']),
              p['core_g_W2'], _row(p['core_g_b2']),
              p['dec_g_W'], _row(p['dec_g_b']),
              w1[384:512], _row(p['core_e_b1']),
              wn1[192:320], _row(p['core_n_b1']),
              p['out_g_W'], _row(p['out_g_b']))

    gsd = _sc_gather(xsd.reshape(2 * NN, H), idx_gather).reshape(EP, 2 * H)
    e2, de, se = _edge0(e_pad, gsd, *enc_e, edge_w[0],
                        edge_w[1], be_b, *edge_tail)
    agg = _sc_scatter(e2, idx_scatter, zeros_np)
    dx, xsd, sx = _node0(lx, agg, *node_w, bn_b, *node_tail)
    dg, be_b, bn_b = _glob0(lg, lg, se, sx, *glob_w)

    gsd = _sc_gather(xsd.reshape(2 * NN, H), idx_gather).reshape(EP, 2 * H)
    e2, de, se, oe = _edge1(e_pad, de, gsd, *enc_e, edge_w[0],
                            edge_w[1], be_b, *edge_tail)
    agg = _sc_scatter(e2, idx_scatter, zeros_np)
    dx, _, sx, ox = _node1(lx, dx, agg, *node_w, bn_b, *node_tail)
    (og,) = _glob1(lg, dg, se, sx, *glob_w)

    return (oe[:NE], ox, og)


# speedup vs baseline: 1.1976x; 1.0735x over previous
import functools

import jax
import jax.numpy as jnp
from jax import lax
from jax.experimental import pallas as pl
from jax.experimental.pallas import tpu as pltpu
from jax.experimental.pallas import tpu_sc as plsc

NN = 10000
NE = 160000
DXD = 128
DED = 16
DGD = 16
H = 64

NTILES = 32
CH = 128
EP = 163840
GR0 = 15360
GR1 = 5120
SCH = EP // NTILES // CH
GCH = 2 * EP // NTILES // CH
NP = 10016
ZR = NP // 16
OR_ = NN // 16

BE = 2048
GE = EP // BE
BN = 2000
GN = NN // BN


@functools.cache
def _sc_kernels():
    mesh = plsc.VectorSubcoreMesh(core_axis_name="c", subcore_axis_name="s")

    nbg = 8

    @functools.partial(
        pl.kernel,
        out_type=jax.ShapeDtypeStruct((2 * EP, H), jnp.float32),
        mesh=mesh,
        compiler_params=pltpu.CompilerParams(use_tc_tiling_on_sc=False),
        scratch_types=[
            pltpu.VMEM((GR0,), jnp.int32),
            pltpu.VMEM((nbg, CH, H), jnp.float32),
        ] + [pltpu.SemaphoreType.DMA] * (2 * nbg),
    )
    def sc_gather(table, idx, out, idx_v, rows_v, *sems):
        gsems, wsems = sems[:nbg], sems[nbg:]
        cid = lax.axis_index("c")
        sid = lax.axis_index("s")

        def run(base, ngroups):
            def gather_src(j):
                return table.at[idx_v.at[pl.ds(j * CH, CH)]]

            def out_dst(j):
                return out.at[pl.ds(base + j * CH, CH)]

            pltpu.sync_copy(idx.at[pl.ds(base, ngroups * nbg * CH)],
                            idx_v.at[pl.ds(0, ngroups * nbg * CH)])

            def body(g, carry):
                for b in range(nbg):
                    j = g * nbg + b

                    @pl.when(g > 0)
                    def _():
                        pltpu.make_async_copy(
                            rows_v.at[b], out_dst(j - nbg), wsems[b]).wait()

                    pltpu.async_copy(gather_src(j), rows_v.at[b], gsems[b])
                for b in range(nbg):
                    j = g * nbg + b
                    pltpu.make_async_copy(gather_src(j), rows_v.at[b],
                                          gsems[b]).wait()
                    pltpu.async_copy(rows_v.at[b], out_dst(j), wsems[b])
                return carry

            lax.fori_loop(0, ngroups, body, 0)
            for b in range(nbg):
                j = (ngroups - 1) * nbg + b
                pltpu.make_async_copy(rows_v.at[b], out_dst(j), wsems[b]).wait()

        @pl.when(cid == 0)
        def _():
            run(sid * GR0, GR0 // CH // nbg)

        @pl.when(cid == 1)
        def _():
            run(16 * GR0 + sid * GR1, GR1 // CH // nbg)

    nbs = 4

    @functools.partial(
        pl.kernel,
        out_type=jax.ShapeDtypeStruct((2, NN, H), jnp.float32),
        mesh=mesh,
        compiler_params=pltpu.CompilerParams(use_tc_tiling_on_sc=False),
        scratch_types=[
            pltpu.VMEM_SHARED((NP, H), jnp.float32),
            pltpu.VMEM((SCH, CH), jnp.int32),
            pltpu.VMEM((nbs, CH, H), jnp.float32),
        ] + [pltpu.SemaphoreType.DMA] * (2 * nbs),
    )
    def sc_scatter(e2, idx3, zeros_hbm, out, shared, idx_v, rows_v, *sems):
        rsems, ssems = sems[:nbs], sems[nbs:]
        cid = lax.axis_index("c")
        sid = lax.axis_index("s")
        wid = sid * 2 + cid
        pltpu.sync_copy(zeros_hbm.at[pl.ds(sid * ZR, ZR)],
                        shared.at[pl.ds(sid * ZR, ZR)])
        pltpu.sync_copy(idx3.at[wid], idx_v)
        plsc.subcore_barrier()
        base = wid * (EP // NTILES)

        def body(g, carry):
            for b in range(nbs):
                j = g * nbs + b

                @pl.when(g > 0)
                def _():
                    pltpu.make_async_copy(
                        rows_v.at[b], shared.at[idx_v.at[j - nbs]], ssems[b]).wait()

                pltpu.async_copy(e2.at[pl.ds(base + j * CH, CH)],
                                 rows_v.at[b], rsems[b])
            for b in range(nbs):
                j = g * nbs + b
                pltpu.make_async_copy(e2.at[pl.ds(base + j * CH, CH)],
                                      rows_v.at[b], rsems[b]).wait()
                pltpu.async_copy(rows_v.at[b], shared.at[idx_v.at[j]],
                                 ssems[b], add=True)
            return carry

        ngroups = SCH // nbs
        lax.fori_loop(0, ngroups, body, 0)
        for b in range(nbs):
            j = (ngroups - 1) * nbs + b
            pltpu.make_async_copy(rows_v.at[b], shared.at[idx_v.at[j]],
                                  ssems[b]).wait()
        plsc.subcore_barrier()
        pltpu.sync_copy(shared.at[pl.ds(sid * OR_, OR_)],
                        out.at[cid, pl.ds(sid * OR_, OR_)])

    return sc_gather, sc_scatter


def _sc_gather(table, idx):
    return _sc_kernels()[0](table, idx)


def _sc_scatter(e2, idx3, zeros_np):
    return _sc_kernels()[1](e2, idx3, zeros_np)


def _ln(h):
    m = jnp.mean(h, axis=-1, keepdims=True)
    v = jnp.var(h, axis=-1, keepdims=True)
    return (h - m) / jnp.sqrt(v + 1e-5)


def _dot(a, b):
    return jax.lax.dot_general(a, b, (((1,), (0,)), ((), ())),
                               preferred_element_type=jnp.float32)


_W64 = pl.BlockSpec((H, H), lambda i: (0, 0))
_B64 = pl.BlockSpec((1, H), lambda i: (0, 0))


def _edge_body(has_de, last, *refs):
    if has_de:
        (e_ref, de_ref, gsd_ref, we, be, w1a, w1b, b1, w2, b2, dw, db, ow, ob,
         e2_ref, de2_ref, sum_ref, *oe) = refs
    else:
        (e_ref, gsd_ref, we, be, w1a, w1b, b1, w2, b2, dw, db, ow, ob,
         e2_ref, de2_ref, sum_ref, *oe) = refs
        de_ref = None
    i = pl.program_id(0)
    le = jnp.maximum(_dot(e_ref[...], we[...]) + be[...], 0.0)
    if has_de:
        dp = de_ref[...]
        de = jnp.concatenate([dp[:, :H], dp[:, H:]], axis=0)
    else:
        de = le
    g = gsd_ref[...]
    h = _dot(le, w1a[...]) + _dot(de, w1b[...])
    h = h + g[:, :H] + g[:, H:] + b1[...]
    h = jnp.maximum(h, 0.0)
    h = jnp.maximum(_dot(h, w2[...]) + b2[...], 0.0)
    e2 = _ln(h)
    e2_ref[...] = jnp.concatenate([e2[:BE // 2], e2[BE // 2:]], axis=1)
    de2 = jnp.maximum(_dot(e2, dw[...]) + db[...], 0.0)
    de2_ref[...] = jnp.concatenate([de2[:BE // 2], de2[BE // 2:]], axis=1)
    rows = i * BE + lax.broadcasted_iota(jnp.int32, (BE, 1), 0)
    part = jnp.sum(jnp.where(rows < NE, e2, 0.0), axis=0, keepdims=True)

    @pl.when(i == 0)
    def _():
        sum_ref[...] = jnp.zeros_like(sum_ref)

    sum_ref[...] += part
    if last:
        oe[0][...] = _dot(de2, ow[...]) + ob[...]


def _make_edge(has_de, last):
    in_specs = [pl.BlockSpec((BE, DED), lambda i: (i, 0))]
    if has_de:
        in_specs.append(pl.BlockSpec((BE // 2, 2 * H), lambda i: (i, 0)))
    in_specs.append(pl.BlockSpec((BE, 2 * H), lambda i: (i, 0)))
    in_specs += [pl.BlockSpec((DED, H), lambda i: (0, 0)), _B64,
                 _W64, _W64, _B64, _W64, _B64, _W64, _B64,
                 pl.BlockSpec((H, DED), lambda i: (0, 0)),
                 pl.BlockSpec((1, DED), lambda i: (0, 0))]
    out_shape = [jax.ShapeDtypeStruct((EP // 2, 2 * H), jnp.float32),
                 jax.ShapeDtypeStruct((EP // 2, 2 * H), jnp.float32),
                 jax.ShapeDtypeStruct((1, H), jnp.float32)]
    out_specs = [pl.BlockSpec((BE // 2, 2 * H), lambda i: (i, 0)),
                 pl.BlockSpec((BE // 2, 2 * H), lambda i: (i, 0)),
                 pl.BlockSpec((1, H), lambda i: (0, 0))]
    if last:
        out_shape.append(jax.ShapeDtypeStruct((EP, DED), jnp.float32))
        out_specs.append(pl.BlockSpec((BE, DED), lambda i: (i, 0)))
    return pl.pallas_call(
        functools.partial(_edge_body, has_de, last),
        grid=(GE,), in_specs=in_specs, out_specs=out_specs, out_shape=out_shape)


_edge0 = _make_edge(False, False)
_edge1 = _make_edge(True, True)


def _node_body(has_dx, last, *refs):
    if has_dx:
        (lx_ref, dx_ref, agg_ref, wn1a, wn1b, wn1c, bn1, wn2, bn2, dw, db,
         wsa, wsb, wda, wdb, ow, ob, dx2_ref, xsd_ref, sum_ref, *ox) = refs
    else:
        (lx_ref, agg_ref, wn1a, wn1b, wn1c, bn1, wn2, bn2, dw, db,
         wsa, wsb, wda, wdb, ow, ob, dx2_ref, xsd_ref, sum_ref, *ox) = refs
        dx_ref = lx_ref
    i = pl.program_id(0)
    lx = lx_ref[...]
    agg = agg_ref[0] + agg_ref[1]
    h = _dot(lx, wn1a[...]) + _dot(dx_ref[...], wn1b[...]) + _dot(agg, wn1c[...]) + bn1[...]
    h = jnp.maximum(h, 0.0)
    h = jnp.maximum(_dot(h, wn2[...]) + bn2[...], 0.0)
    x2 = _ln(h)
    dx2 = jnp.maximum(_dot(x2, dw[...]) + db[...], 0.0)
    dx2_ref[...] = dx2
    xsd_ref[0] = _dot(lx, wsa[...]) + _dot(dx2, wsb[...])
    xsd_ref[1] = _dot(lx, wda[...]) + _dot(dx2, wdb[...])
    part = jnp.sum(x2, axis=0, keepdims=True)

    @pl.when(i == 0)
    def _():
        sum_ref[...] = jnp.zeros_like(sum_ref)

    sum_ref[...] += part
    if last:
        ox[0][...] = _dot(dx2, ow[...]) + ob[...]


def _make_node(has_dx, last):
    in_specs = [pl.BlockSpec((BN, H), lambda i: (i, 0))]
    if has_dx:
        in_specs.append(pl.BlockSpec((BN, H), lambda i: (i, 0)))
    in_specs.append(pl.BlockSpec((2, BN, H), lambda i: (0, i, 0)))
    in_specs += [_W64, _W64, _W64, _B64, _W64, _B64, _W64, _B64,
                 _W64, _W64, _W64, _W64,
                 pl.BlockSpec((H, DXD), lambda i: (0, 0)),
                 pl.BlockSpec((1, DXD), lambda i: (0, 0))]
    out_shape = [jax.ShapeDtypeStruct((NN, H), jnp.float32),
                 jax.ShapeDtypeStruct((2, NN, H), jnp.float32),
                 jax.ShapeDtypeStruct((1, H), jnp.float32)]
    out_specs = [pl.BlockSpec((BN, H), lambda i: (i, 0)),
                 pl.BlockSpec((2, BN, H), lambda i: (0, i, 0)),
                 pl.BlockSpec((1, H), lambda i: (0, 0))]
    if last:
        out_shape.append(jax.ShapeDtypeStruct((NN, DXD), jnp.float32))
        out_specs.append(pl.BlockSpec((BN, DXD), lambda i: (i, 0)))
    return pl.pallas_call(
        functools.partial(_node_body, has_dx, last),
        grid=(GN,), in_specs=in_specs, out_specs=out_specs, out_shape=out_shape)


_node0 = _make_node(False, False)
_node1 = _make_node(True, True)


def _enc_e_body(e_ref, w_ref, b_ref, le_ref):
    le_ref[...] = jnp.maximum(_dot(e_ref[...], w_ref[...]) + b_ref[...], 0.0)


_enc_e = pl.pallas_call(
    _enc_e_body, grid=(GE,),
    in_specs=[pl.BlockSpec((BE, DED), lambda i: (i, 0)),
              pl.BlockSpec((DED, H), lambda i: (0, 0)),
              pl.BlockSpec((1, H), lambda i: (0, 0))],
    out_specs=pl.BlockSpec((BE, H), lambda i: (i, 0)),
    out_shape=jax.ShapeDtypeStruct((EP, H), jnp.float32))


def _enc_x_body(x_ref, w_ref, b_ref, ws_ref, wd_ref, lx_ref, xsd_ref):
    lx = jnp.maximum(_dot(x_ref[...], w_ref[...]) + b_ref[...], 0.0)
    lx_ref[...] = lx
    lxc = jnp.concatenate([lx, lx], axis=1)
    xsd_ref[0] = _dot(lxc, ws_ref[...])
    xsd_ref[1] = _dot(lxc, wd_ref[...])


_enc_x = pl.pallas_call(
    _enc_x_body, grid=(GN,),
    in_specs=[pl.BlockSpec((BN, DXD), lambda i: (i, 0)),
              pl.BlockSpec((DXD, H), lambda i: (0, 0)),
              pl.BlockSpec((1, H), lambda i: (0, 0)),
              pl.BlockSpec((2 * H, H), lambda i: (0, 0)),
              pl.BlockSpec((2 * H, H), lambda i: (0, 0))],
    out_specs=[pl.BlockSpec((BN, H), lambda i: (i, 0)),
               pl.BlockSpec((2, BN, H), lambda i: (0, i, 0))],
    out_shape=[jax.ShapeDtypeStruct((NN, H), jnp.float32),
               jax.ShapeDtypeStruct((2, NN, H), jnp.float32)])


def _prep_g_body(g_ref, w_ref, b_ref, wge_ref, b1e_ref, wgn_ref, b1n_ref,
                 lg_ref, be_ref, bn_ref):
    lg = jnp.maximum(_dot(g_ref[...], w_ref[...]) + b_ref[...], 0.0)
    lg_ref[...] = lg
    lgc = jnp.concatenate([lg, lg], axis=1)
    be_ref[...] = _dot(lgc, wge_ref[...]) + b1e_ref[...]
    bn_ref[...] = _dot(lgc, wgn_ref[...]) + b1n_ref[...]


_prep_g = pl.pallas_call(
    _prep_g_body,
    out_shape=[jax.ShapeDtypeStruct((1, H), jnp.float32)] * 3)


def _glob_body(last, *refs):
    (lg_ref, dg_ref, se_ref, sx_ref, wg1, bg1, wg2, bg2, dw, db,
     wge, b1e, wgn, b1n, ow, ob, *outs) = refs
    gcat = jnp.concatenate([lg_ref[...], dg_ref[...]], axis=1)
    gin = jnp.concatenate([gcat, se_ref[...], sx_ref[...]], axis=1)
    h = jnp.maximum(_dot(gin, wg1[...]) + bg1[...], 0.0)
    h = jnp.maximum(_dot(h, wg2[...]) + bg2[...], 0.0)
    g2 = _ln(h)
    dg2 = jnp.maximum(_dot(g2, dw[...]) + db[...], 0.0)
    if last:
        outs[0][...] = _dot(dg2, ow[...]) + ob[...]
    else:
        dg_out, be_out, bn_out = outs
        dg_out[...] = dg2
        gcat2 = jnp.concatenate([lg_ref[...], dg2], axis=1)
        be_out[...] = _dot(gcat2, wge[...]) + b1e[...]
        bn_out[...] = _dot(gcat2, wgn[...]) + b1n[...]


_glob0 = pl.pallas_call(
    functools.partial(_glob_body, False),
    out_shape=[jax.ShapeDtypeStruct((1, H), jnp.float32)] * 3)
_glob1 = pl.pallas_call(
    functools.partial(_glob_body, True),
    out_shape=[jax.ShapeDtypeStruct((1, DGD), jnp.float32)])


def _row(b):
    return b[None, :]


def kernel(x, e, g, params, edges, node_idx, edge_idx, steps):
    p = params
    src = edges[0]
    dst = edges[1]
    padn = EP - NE
    zpad = jnp.zeros((padn,), jnp.int32)
    idx_gather = jnp.stack(
        [jnp.concatenate([src, zpad]),
         jnp.concatenate([dst + NN, zpad])], axis=1).reshape(-1)
    idx_scatter = jnp.concatenate(
        [dst, jnp.full((padn,), NN, jnp.int32)]).reshape(NTILES, SCH, CH)
    zeros_np = jnp.zeros((NP, H), jnp.float32)
    e_pad = jnp.pad(e, ((0, padn), (0, 0)))
    r = jnp.arange(EP, dtype=jnp.int32)
    perm = (r // BE) * BE + (r % 2) * (BE // 2) + (r % BE) // 2
    idx_scatter = jnp.take(idx_scatter.reshape(-1), perm).reshape(
        NTILES, SCH, CH)

    w1 = p['core_e_W1']
    wn1 = p['core_n_W1']

    lx, xsd = _enc_x(x, p['enc_x_W'], _row(p['enc_x_b']),
                     w1[128:256], w1[256:384])
    lg, be_b, bn_b = _prep_g(g, p['enc_g_W'], _row(p['enc_g_b']),
                             w1[384:512], _row(p['core_e_b1']),
                             wn1[192:320], _row(p['core_n_b1']))

    enc_e = (p['enc_e_W'], _row(p['enc_e_b']))
    edge_w = (w1[0:64], w1[64:128])
    edge_tail = (p['core_e_W2'], _row(p['core_e_b2']),
                 p['dec_e_W'], _row(p['dec_e_b']),
                 p['out_e_W'], _row(p['out_e_b']))
    node_w = (wn1[0:64], wn1[64:128], wn1[128:192])
    node_tail = (p['core_n_W2'], _row(p['core_n_b2']),
                 p['dec_x_W'], _row(p['dec_x_b']),
                 w1[128:192], w1[192:256], w1[256:320], w1[320:384],
                 p['out_x_W'], _row(p['out_x_b']))
    glob_w = (p['core_g_W1'], _row(p['core_g_b1']),
              p['core_g_W2'], _row(p['core_g_b2']),
              p['dec_g_W'], _row(p['dec_g_b']),
              w1[384:512], _row(p['core_e_b1']),
              wn1[192:320], _row(p['core_n_b1']),
              p['out_g_W'], _row(p['out_g_b']))

    gsd = _sc_gather(xsd.reshape(2 * NN, H), idx_gather).reshape(EP, 2 * H)
    e2, de, se = _edge0(e_pad, gsd, *enc_e, edge_w[0],
                        edge_w[1], be_b, *edge_tail)
    agg = _sc_scatter(e2.reshape(EP, H), idx_scatter, zeros_np)
    dx, xsd, sx = _node0(lx, agg, *node_w, bn_b, *node_tail)
    dg, be_b, bn_b = _glob0(lg, lg, se, sx, *glob_w)

    gsd = _sc_gather(xsd.reshape(2 * NN, H), idx_gather).reshape(EP, 2 * H)
    e2, de, se, oe = _edge1(e_pad, de, gsd, *enc_e, edge_w[0],
                            edge_w[1], be_b, *edge_tail)
    agg = _sc_scatter(e2.reshape(EP, H), idx_scatter, zeros_np)
    dx, _, sx, ox = _node1(lx, dx, agg, *node_w, bn_b, *node_tail)
    (og,) = _glob1(lg, dg, se, sx, *glob_w)

    return (oe[:NE], ox, og)


# speedup vs baseline: 1.2162x; 1.0156x over previous
import functools

import jax
import jax.numpy as jnp
from jax import lax
from jax.experimental import pallas as pl
from jax.experimental.pallas import tpu as pltpu
from jax.experimental.pallas import tpu_sc as plsc

NN = 10000
NE = 160000
DXD = 128
DED = 16
DGD = 16
H = 64

NTILES = 32
CH = 128
EP = 163840
GR0 = 20480
SCH = EP // NTILES // CH
GCH = 2 * EP // NTILES // CH
NP = 10016
ZR = NP // 16
OR_ = NN // 16

BE = 2048
GE = EP // BE
BN = 2000
GN = NN // BN


@functools.cache
def _sc_kernels():
    mesh = plsc.VectorSubcoreMesh(core_axis_name="c", subcore_axis_name="s")

    nbg = 8

    @functools.partial(
        pl.kernel,
        out_type=jax.ShapeDtypeStruct((2 * EP, H), jnp.float32),
        mesh=mesh,
        compiler_params=pltpu.CompilerParams(use_tc_tiling_on_sc=False),
        scratch_types=[
            pltpu.VMEM((GR0,), jnp.int32),
            pltpu.VMEM((nbg, CH, H), jnp.float32),
        ] + [pltpu.SemaphoreType.DMA] * (2 * nbg),
    )
    def sc_gather(table, idx, out, idx_v, rows_v, *sems):
        gsems, wsems = sems[:nbg], sems[nbg:]
        cid = lax.axis_index("c")
        sid = lax.axis_index("s")

        def run(base, ngroups):
            def gather_src(j):
                return table.at[idx_v.at[pl.ds(j * CH, CH)]]

            def out_dst(j):
                return out.at[pl.ds(base + j * CH, CH)]

            pltpu.sync_copy(idx.at[pl.ds(base, ngroups * nbg * CH)],
                            idx_v.at[pl.ds(0, ngroups * nbg * CH)])

            def body(g, carry):
                for b in range(nbg):
                    j = g * nbg + b

                    @pl.when(g > 0)
                    def _():
                        pltpu.make_async_copy(
                            rows_v.at[b], out_dst(j - nbg), wsems[b]).wait()

                    pltpu.async_copy(gather_src(j), rows_v.at[b], gsems[b])
                for b in range(nbg):
                    j = g * nbg + b
                    pltpu.make_async_copy(gather_src(j), rows_v.at[b],
                                          gsems[b]).wait()
                    pltpu.async_copy(rows_v.at[b], out_dst(j), wsems[b])
                return carry

            lax.fori_loop(0, ngroups, body, 0)
            for b in range(nbg):
                j = (ngroups - 1) * nbg + b
                pltpu.make_async_copy(rows_v.at[b], out_dst(j), wsems[b]).wait()

        @pl.when(cid == 0)
        def _():
            run(sid * GR0, GR0 // CH // nbg)

    nbs = 4

    @functools.partial(
        pl.kernel,
        out_type=jax.ShapeDtypeStruct((2, NN, H), jnp.float32),
        mesh=mesh,
        compiler_params=pltpu.CompilerParams(use_tc_tiling_on_sc=False),
        scratch_types=[
            pltpu.VMEM_SHARED((NP, H), jnp.float32),
            pltpu.VMEM((SCH, CH), jnp.int32),
            pltpu.VMEM((nbs, CH, H), jnp.float32),
        ] + [pltpu.SemaphoreType.DMA] * (2 * nbs),
    )
    def sc_scatter(e2, idx3, zeros_hbm, out, shared, idx_v, rows_v, *sems):
        rsems, ssems = sems[:nbs], sems[nbs:]
        cid = lax.axis_index("c")
        sid = lax.axis_index("s")
        wid = sid * 2 + cid
        pltpu.sync_copy(zeros_hbm.at[pl.ds(sid * ZR, ZR)],
                        shared.at[pl.ds(sid * ZR, ZR)])
        pltpu.sync_copy(idx3.at[wid], idx_v)
        plsc.subcore_barrier()
        base = wid * (EP // NTILES)

        def body(g, carry):
            for b in range(nbs):
                j = g * nbs + b

                @pl.when(g > 0)
                def _():
                    pltpu.make_async_copy(
                        rows_v.at[b], shared.at[idx_v.at[j - nbs]], ssems[b]).wait()

                pltpu.async_copy(e2.at[pl.ds(base + j * CH, CH)],
                                 rows_v.at[b], rsems[b])
            for b in range(nbs):
                j = g * nbs + b
                pltpu.make_async_copy(e2.at[pl.ds(base + j * CH, CH)],
                                      rows_v.at[b], rsems[b]).wait()
                pltpu.async_copy(rows_v.at[b], shared.at[idx_v.at[j]],
                                 ssems[b], add=True)
            return carry

        ngroups = SCH // nbs
        lax.fori_loop(0, ngroups, body, 0)
        for b in range(nbs):
            j = (ngroups - 1) * nbs + b
            pltpu.make_async_copy(rows_v.at[b], shared.at[idx_v.at[j]],
                                  ssems[b]).wait()
        plsc.subcore_barrier()
        pltpu.sync_copy(shared.at[pl.ds(sid * OR_, OR_)],
                        out.at[cid, pl.ds(sid * OR_, OR_)])

    return sc_gather, sc_scatter


def _sc_gather(table, idx):
    return _sc_kernels()[0](table, idx)


def _sc_scatter(e2, idx3, zeros_np):
    return _sc_kernels()[1](e2, idx3, zeros_np)


def _ln(h):
    m = jnp.mean(h, axis=-1, keepdims=True)
    v = jnp.var(h, axis=-1, keepdims=True)
    return (h - m) / jnp.sqrt(v + 1e-5)


def _dot(a, b):
    return jax.lax.dot_general(a, b, (((1,), (0,)), ((), ())),
                               preferred_element_type=jnp.float32)


_W64 = pl.BlockSpec((H, H), lambda i: (0, 0))
_B64 = pl.BlockSpec((1, H), lambda i: (0, 0))


def _edge_body(has_de, last, *refs):
    if has_de:
        (e_ref, de_ref, gsd_ref, we, be, w1a, w1b, b1, w2, b2, dw, db, ow, ob,
         e2_ref, de2_ref, sum_ref, *oe) = refs
    else:
        (e_ref, gsd_ref, we, be, w1a, w1b, b1, w2, b2, dw, db, ow, ob,
         e2_ref, de2_ref, sum_ref, *oe) = refs
        de_ref = None
    i = pl.program_id(0)
    le = jnp.maximum(_dot(e_ref[...], we[...]) + be[...], 0.0)
    if has_de:
        dp = de_ref[...]
        de = jnp.concatenate([dp[:, :H], dp[:, H:]], axis=0)
    else:
        de = le
    g = gsd_ref[...]
    h = _dot(le, w1a[...]) + _dot(de, w1b[...])
    h = h + g[:, :H] + g[:, H:] + b1[...]
    h = jnp.maximum(h, 0.0)
    h = jnp.maximum(_dot(h, w2[...]) + b2[...], 0.0)
    e2 = _ln(h)
    e2_ref[...] = jnp.concatenate([e2[:BE // 2], e2[BE // 2:]], axis=1)
    de2 = jnp.maximum(_dot(e2, dw[...]) + db[...], 0.0)
    de2_ref[...] = jnp.concatenate([de2[:BE // 2], de2[BE // 2:]], axis=1)
    rows = i * BE + lax.broadcasted_iota(jnp.int32, (BE, 1), 0)
    part = jnp.sum(jnp.where(rows < NE, e2, 0.0), axis=0, keepdims=True)

    @pl.when(i == 0)
    def _():
        sum_ref[...] = jnp.zeros_like(sum_ref)

    sum_ref[...] += part
    if last:
        oe[0][...] = _dot(de2, ow[...]) + ob[...]


def _make_edge(has_de, last):
    in_specs = [pl.BlockSpec((BE, DED), lambda i: (i, 0))]
    if has_de:
        in_specs.append(pl.BlockSpec((BE // 2, 2 * H), lambda i: (i, 0)))
    in_specs.append(pl.BlockSpec((BE, 2 * H), lambda i: (i, 0)))
    in_specs += [pl.BlockSpec((DED, H), lambda i: (0, 0)), _B64,
                 _W64, _W64, _B64, _W64, _B64, _W64, _B64,
                 pl.BlockSpec((H, DED), lambda i: (0, 0)),
                 pl.BlockSpec((1, DED), lambda i: (0, 0))]
    out_shape = [jax.ShapeDtypeStruct((EP // 2, 2 * H), jnp.float32),
                 jax.ShapeDtypeStruct((EP // 2, 2 * H), jnp.float32),
                 jax.ShapeDtypeStruct((1, H), jnp.float32)]
    out_specs = [pl.BlockSpec((BE // 2, 2 * H), lambda i: (i, 0)),
                 pl.BlockSpec((BE // 2, 2 * H), lambda i: (i, 0)),
                 pl.BlockSpec((1, H), lambda i: (0, 0))]
    if last:
        out_shape.append(jax.ShapeDtypeStruct((EP, DED), jnp.float32))
        out_specs.append(pl.BlockSpec((BE, DED), lambda i: (i, 0)))
    return pl.pallas_call(
        functools.partial(_edge_body, has_de, last),
        grid=(GE,), in_specs=in_specs, out_specs=out_specs, out_shape=out_shape)


_edge0 = _make_edge(False, False)
_edge1 = _make_edge(True, True)


def _node_body(has_dx, last, *refs):
    if has_dx:
        (lx_ref, dx_ref, agg_ref, wn1a, wn1b, wn1c, bn1, wn2, bn2, dw, db,
         wsa, wsb, wda, wdb, ow, ob, dx2_ref, xsd_ref, sum_ref, *ox) = refs
    else:
        (lx_ref, agg_ref, wn1a, wn1b, wn1c, bn1, wn2, bn2, dw, db,
         wsa, wsb, wda, wdb, ow, ob, dx2_ref, xsd_ref, sum_ref, *ox) = refs
        dx_ref = lx_ref
    i = pl.program_id(0)
    lx = lx_ref[...]
    agg = agg_ref[0] + agg_ref[1]
    h = _dot(lx, wn1a[...]) + _dot(dx_ref[...], wn1b[...]) + _dot(agg, wn1c[...]) + bn1[...]
    h = jnp.maximum(h, 0.0)
    h = jnp.maximum(_dot(h, wn2[...]) + bn2[...], 0.0)
    x2 = _ln(h)
    dx2 = jnp.maximum(_dot(x2, dw[...]) + db[...], 0.0)
    dx2_ref[...] = dx2
    xsd_ref[...] = jnp.concatenate(
        [_dot(lx, wsa[...]) + _dot(dx2, wsb[...]),
         _dot(lx, wda[...]) + _dot(dx2, wdb[...])], axis=1)
    part = jnp.sum(x2, axis=0, keepdims=True)

    @pl.when(i == 0)
    def _():
        sum_ref[...] = jnp.zeros_like(sum_ref)

    sum_ref[...] += part
    if last:
        ox[0][...] = _dot(dx2, ow[...]) + ob[...]


def _make_node(has_dx, last):
    in_specs = [pl.BlockSpec((BN, H), lambda i: (i, 0))]
    if has_dx:
        in_specs.append(pl.BlockSpec((BN, H), lambda i: (i, 0)))
    in_specs.append(pl.BlockSpec((2, BN, H), lambda i: (0, i, 0)))
    in_specs += [_W64, _W64, _W64, _B64, _W64, _B64, _W64, _B64,
                 _W64, _W64, _W64, _W64,
                 pl.BlockSpec((H, DXD), lambda i: (0, 0)),
                 pl.BlockSpec((1, DXD), lambda i: (0, 0))]
    out_shape = [jax.ShapeDtypeStruct((NN, H), jnp.float32),
                 jax.ShapeDtypeStruct((NN, 2 * H), jnp.float32),
                 jax.ShapeDtypeStruct((1, H), jnp.float32)]
    out_specs = [pl.BlockSpec((BN, H), lambda i: (i, 0)),
                 pl.BlockSpec((BN, 2 * H), lambda i: (i, 0)),
                 pl.BlockSpec((1, H), lambda i: (0, 0))]
    if last:
        out_shape.append(jax.ShapeDtypeStruct((NN, DXD), jnp.float32))
        out_specs.append(pl.BlockSpec((BN, DXD), lambda i: (i, 0)))
    return pl.pallas_call(
        functools.partial(_node_body, has_dx, last),
        grid=(GN,), in_specs=in_specs, out_specs=out_specs, out_shape=out_shape)


_node0 = _make_node(False, False)
_node1 = _make_node(True, True)


def _enc_e_body(e_ref, w_ref, b_ref, le_ref):
    le_ref[...] = jnp.maximum(_dot(e_ref[...], w_ref[...]) + b_ref[...], 0.0)


_enc_e = pl.pallas_call(
    _enc_e_body, grid=(GE,),
    in_specs=[pl.BlockSpec((BE, DED), lambda i: (i, 0)),
              pl.BlockSpec((DED, H), lambda i: (0, 0)),
              pl.BlockSpec((1, H), lambda i: (0, 0))],
    out_specs=pl.BlockSpec((BE, H), lambda i: (i, 0)),
    out_shape=jax.ShapeDtypeStruct((EP, H), jnp.float32))


def _enc_x_body(x_ref, w_ref, b_ref, ws_ref, wd_ref, lx_ref, xsd_ref):
    lx = jnp.maximum(_dot(x_ref[...], w_ref[...]) + b_ref[...], 0.0)
    lx_ref[...] = lx
    lxc = jnp.concatenate([lx, lx], axis=1)
    xsd_ref[...] = jnp.concatenate(
        [_dot(lxc, ws_ref[...]), _dot(lxc, wd_ref[...])], axis=1)


_enc_x = pl.pallas_call(
    _enc_x_body, grid=(GN,),
    in_specs=[pl.BlockSpec((BN, DXD), lambda i: (i, 0)),
              pl.BlockSpec((DXD, H), lambda i: (0, 0)),
              pl.BlockSpec((1, H), lambda i: (0, 0)),
              pl.BlockSpec((2 * H, H), lambda i: (0, 0)),
              pl.BlockSpec((2 * H, H), lambda i: (0, 0))],
    out_specs=[pl.BlockSpec((BN, H), lambda i: (i, 0)),
               pl.BlockSpec((BN, 2 * H), lambda i: (i, 0))],
    out_shape=[jax.ShapeDtypeStruct((NN, H), jnp.float32),
               jax.ShapeDtypeStruct((NN, 2 * H), jnp.float32)])


def _prep_g_body(g_ref, w_ref, b_ref, wge_ref, b1e_ref, wgn_ref, b1n_ref,
                 lg_ref, be_ref, bn_ref):
    lg = jnp.maximum(_dot(g_ref[...], w_ref[...]) + b_ref[...], 0.0)
    lg_ref[...] = lg
    lgc = jnp.concatenate([lg, lg], axis=1)
    be_ref[...] = _dot(lgc, wge_ref[...]) + b1e_ref[...]
    bn_ref[...] = _dot(lgc, wgn_ref[...]) + b1n_ref[...]


_prep_g = pl.pallas_call(
    _prep_g_body,
    out_shape=[jax.ShapeDtypeStruct((1, H), jnp.float32)] * 3)


def _glob_body(last, *refs):
    (lg_ref, dg_ref, se_ref, sx_ref, wg1, bg1, wg2, bg2, dw, db,
     wge, b1e, wgn, b1n, ow, ob, *outs) = refs
    gcat = jnp.concatenate([lg_ref[...], dg_ref[...]], axis=1)
    gin = jnp.concatenate([gcat, se_ref[...], sx_ref[...]], axis=1)
    h = jnp.maximum(_dot(gin, wg1[...]) + bg1[...], 0.0)
    h = jnp.maximum(_dot(h, wg2[...]) + bg2[...], 0.0)
    g2 = _ln(h)
    dg2 = jnp.maximum(_dot(g2, dw[...]) + db[...], 0.0)
    if last:
        outs[0][...] = _dot(dg2, ow[...]) + ob[...]
    else:
        dg_out, be_out, bn_out = outs
        dg_out[...] = dg2
        gcat2 = jnp.concatenate([lg_ref[...], dg2], axis=1)
        be_out[...] = _dot(gcat2, wge[...]) + b1e[...]
        bn_out[...] = _dot(gcat2, wgn[...]) + b1n[...]


_glob0 = pl.pallas_call(
    functools.partial(_glob_body, False),
    out_shape=[jax.ShapeDtypeStruct((1, H), jnp.float32)] * 3)
_glob1 = pl.pallas_call(
    functools.partial(_glob_body, True),
    out_shape=[jax.ShapeDtypeStruct((1, DGD), jnp.float32)])


def _row(b):
    return b[None, :]


def kernel(x, e, g, params, edges, node_idx, edge_idx, steps):
    p = params
    src = edges[0]
    dst = edges[1]
    padn = EP - NE
    zpad = jnp.zeros((padn,), jnp.int32)
    idx_gather = jnp.stack(
        [2 * jnp.concatenate([src, zpad]),
         2 * jnp.concatenate([dst, zpad]) + 1], axis=1).reshape(-1)
    idx_scatter = jnp.concatenate(
        [dst, jnp.full((padn,), NN, jnp.int32)]).reshape(NTILES, SCH, CH)
    zeros_np = jnp.zeros((NP, H), jnp.float32)
    e_pad = jnp.pad(e, ((0, padn), (0, 0)))
    r = jnp.arange(EP, dtype=jnp.int32)
    perm = (r // BE) * BE + (r % 2) * (BE // 2) + (r % BE) // 2
    idx_scatter = jnp.take(idx_scatter.reshape(-1), perm).reshape(
        NTILES, SCH, CH)

    w1 = p['core_e_W1']
    wn1 = p['core_n_W1']

    lx, xsd = _enc_x(x, p['enc_x_W'], _row(p['enc_x_b']),
                     w1[128:256], w1[256:384])
    lg, be_b, bn_b = _prep_g(g, p['enc_g_W'], _row(p['enc_g_b']),
                             w1[384:512], _row(p['core_e_b1']),
                             wn1[192:320], _row(p['core_n_b1']))

    enc_e = (p['enc_e_W'], _row(p['enc_e_b']))
    edge_w = (w1[0:64], w1[64:128])
    edge_tail = (p['core_e_W2'], _row(p['core_e_b2']),
                 p['dec_e_W'], _row(p['dec_e_b']),
                 p['out_e_W'], _row(p['out_e_b']))
    node_w = (wn1[0:64], wn1[64:128], wn1[128:192])
    node_tail = (p['core_n_W2'], _row(p['core_n_b2']),
                 p['dec_x_W'], _row(p['dec_x_b']),
                 w1[128:192], w1[192:256], w1[256:320], w1[320:384],
                 p['out_x_W'], _row(p['out_x_b']))
    glob_w = (p['core_g_W1'], _row(p['core_g_b1']),
              p['core_g_W2'], _row(p['core_g_b2']),
              p['dec_g_W'], _row(p['dec_g_b']),
              w1[384:512], _row(p['core_e_b1']),
              wn1[192:320], _row(p['core_n_b1']),
              p['out_g_W'], _row(p['out_g_b']))

    gsd = _sc_gather(xsd.reshape(2 * NN, H), idx_gather).reshape(EP, 2 * H)
    e2, de, se = _edge0(e_pad, gsd, *enc_e, edge_w[0],
                        edge_w[1], be_b, *edge_tail)
    agg = _sc_scatter(e2.reshape(EP, H), idx_scatter, zeros_np)
    dx, xsd, sx = _node0(lx, agg, *node_w, bn_b, *node_tail)
    dg, be_b, bn_b = _glob0(lg, lg, se, sx, *glob_w)

    gsd = _sc_gather(xsd.reshape(2 * NN, H), idx_gather).reshape(EP, 2 * H)
    e2, de, se, oe = _edge1(e_pad, de, gsd, *enc_e, edge_w[0],
                            edge_w[1], be_b, *edge_tail)
    agg = _sc_scatter(e2.reshape(EP, H), idx_scatter, zeros_np)
    dx, _, sx, ox = _node1(lx, dx, agg, *node_w, bn_b, *node_tail)
    (og,) = _glob1(lg, dg, se, sx, *glob_w)

    return (oe[:NE], ox, og)


# speedup vs baseline: 1.9126x; 1.5726x over previous
import functools

import jax
import jax.numpy as jnp
from jax import lax
from jax.experimental import pallas as pl
from jax.experimental.pallas import tpu as pltpu
from jax.experimental.pallas import tpu_sc as plsc

NN = 10000
NE = 160000
DXD = 128
DED = 16
DGD = 16
H = 64

NTILES = 32
CH = 128
EP = 163840
SCH = EP // NTILES // CH
GCH = 2 * EP // NTILES // CH
NP = 10016
ZR = NP // 16
OR_ = NN // 16

BE = 2048
GE = EP // BE
BN = 2000
GN = NN // BN


@functools.cache
def _sc_kernels():
    mesh = plsc.VectorSubcoreMesh(core_axis_name="c", subcore_axis_name="s")

    nbg = 4

    @functools.partial(
        pl.kernel,
        out_type=jax.ShapeDtypeStruct((2 * EP, H), jnp.float32),
        mesh=mesh,
        compiler_params=pltpu.CompilerParams(use_tc_tiling_on_sc=False),
        scratch_types=[
            pltpu.VMEM_SHARED((2 * NN, H), jnp.float32),
            pltpu.VMEM((2 * EP // NTILES,), jnp.int32),
            pltpu.VMEM((nbg, CH, H), jnp.float32),
        ] + [pltpu.SemaphoreType.DMA] * (2 * nbg),
    )
    def sc_gather(table, idx, out, shared_t, idx_v, rows_v, *sems):
        gsems, wsems = sems[:nbg], sems[nbg:]
        cid = lax.axis_index("c")
        sid = lax.axis_index("s")
        trows = 2 * NN // 16
        pltpu.sync_copy(table.at[pl.ds(sid * trows, trows)],
                        shared_t.at[pl.ds(sid * trows, trows)])
        plsc.subcore_barrier()

        def run(base, ngroups):
            def gather_src(j):
                return shared_t.at[idx_v.at[pl.ds(j * CH, CH)]]

            def out_dst(j):
                return out.at[pl.ds(base + j * CH, CH)]

            pltpu.sync_copy(idx.at[pl.ds(base, ngroups * nbg * CH)],
                            idx_v.at[pl.ds(0, ngroups * nbg * CH)])

            def body(g, carry):
                for b in range(nbg):
                    j = g * nbg + b

                    @pl.when(g > 0)
                    def _():
                        pltpu.make_async_copy(
                            rows_v.at[b], out_dst(j - nbg), wsems[b]).wait()

                    pltpu.async_copy(gather_src(j), rows_v.at[b], gsems[b])
                for b in range(nbg):
                    j = g * nbg + b
                    pltpu.make_async_copy(gather_src(j), rows_v.at[b],
                                          gsems[b]).wait()
                    pltpu.async_copy(rows_v.at[b], out_dst(j), wsems[b])
                return carry

            lax.fori_loop(0, ngroups, body, 0)
            for b in range(nbg):
                j = (ngroups - 1) * nbg + b
                pltpu.make_async_copy(rows_v.at[b], out_dst(j), wsems[b]).wait()

        wid = sid * 2 + cid
        run(wid * (2 * EP // NTILES), 2 * EP // NTILES // CH // nbg)

    nbs = 4

    @functools.partial(
        pl.kernel,
        out_type=jax.ShapeDtypeStruct((2, NN, H), jnp.float32),
        mesh=mesh,
        compiler_params=pltpu.CompilerParams(use_tc_tiling_on_sc=False),
        scratch_types=[
            pltpu.VMEM_SHARED((NP, H), jnp.float32),
            pltpu.VMEM((SCH, CH), jnp.int32),
            pltpu.VMEM((nbs, CH, H), jnp.float32),
        ] + [pltpu.SemaphoreType.DMA] * (2 * nbs),
    )
    def sc_scatter(e2, idx3, zeros_hbm, out, shared, idx_v, rows_v, *sems):
        rsems, ssems = sems[:nbs], sems[nbs:]
        cid = lax.axis_index("c")
        sid = lax.axis_index("s")
        wid = sid * 2 + cid
        pltpu.sync_copy(zeros_hbm.at[pl.ds(sid * ZR, ZR)],
                        shared.at[pl.ds(sid * ZR, ZR)])
        pltpu.sync_copy(idx3.at[wid], idx_v)
        plsc.subcore_barrier()
        base = wid * (EP // NTILES)

        def body(g, carry):
            for b in range(nbs):
                j = g * nbs + b

                @pl.when(g > 0)
                def _():
                    pltpu.make_async_copy(
                        rows_v.at[b], shared.at[idx_v.at[j - nbs]], ssems[b]).wait()

                pltpu.async_copy(e2.at[pl.ds(base + j * CH, CH)],
                                 rows_v.at[b], rsems[b])
            for b in range(nbs):
                j = g * nbs + b
                pltpu.make_async_copy(e2.at[pl.ds(base + j * CH, CH)],
                                      rows_v.at[b], rsems[b]).wait()
                pltpu.async_copy(rows_v.at[b], shared.at[idx_v.at[j]],
                                 ssems[b], add=True)
            return carry

        ngroups = SCH // nbs
        lax.fori_loop(0, ngroups, body, 0)
        for b in range(nbs):
            j = (ngroups - 1) * nbs + b
            pltpu.make_async_copy(rows_v.at[b], shared.at[idx_v.at[j]],
                                  ssems[b]).wait()
        plsc.subcore_barrier()
        pltpu.sync_copy(shared.at[pl.ds(sid * OR_, OR_)],
                        out.at[cid, pl.ds(sid * OR_, OR_)])

    return sc_gather, sc_scatter


def _sc_gather(table, idx):
    return _sc_kernels()[0](table, idx)


def _sc_scatter(e2, idx3, zeros_np):
    return _sc_kernels()[1](e2, idx3, zeros_np)


def _ln(h):
    m = jnp.mean(h, axis=-1, keepdims=True)
    v = jnp.var(h, axis=-1, keepdims=True)
    return (h - m) / jnp.sqrt(v + 1e-5)


def _dot(a, b):
    return jax.lax.dot_general(a, b, (((1,), (0,)), ((), ())),
                               preferred_element_type=jnp.float32)


_W64 = pl.BlockSpec((H, H), lambda i: (0, 0))
_B64 = pl.BlockSpec((1, H), lambda i: (0, 0))


def _edge_body(has_de, last, *refs):
    if has_de:
        (e_ref, de_ref, gsd_ref, we, be, w1a, w1b, b1, w2, b2, dw, db, ow, ob,
         e2_ref, de2_ref, sum_ref, *oe) = refs
    else:
        (e_ref, gsd_ref, we, be, w1a, w1b, b1, w2, b2, dw, db, ow, ob,
         e2_ref, de2_ref, sum_ref, *oe) = refs
        de_ref = None
    i = pl.program_id(0)
    le = jnp.maximum(_dot(e_ref[...], we[...]) + be[...], 0.0)
    if has_de:
        dp = de_ref[...]
        de = jnp.concatenate([dp[:, :H], dp[:, H:]], axis=0)
    else:
        de = le
    g = gsd_ref[...]
    h = _dot(le, w1a[...]) + _dot(de, w1b[...])
    h = h + g[:, :H] + g[:, H:] + b1[...]
    h = jnp.maximum(h, 0.0)
    h = jnp.maximum(_dot(h, w2[...]) + b2[...], 0.0)
    e2 = _ln(h)
    e2_ref[...] = jnp.concatenate([e2[:BE // 2], e2[BE // 2:]], axis=1)
    de2 = jnp.maximum(_dot(e2, dw[...]) + db[...], 0.0)
    de2_ref[...] = jnp.concatenate([de2[:BE // 2], de2[BE // 2:]], axis=1)
    rows = i * BE + lax.broadcasted_iota(jnp.int32, (BE, 1), 0)
    part = jnp.sum(jnp.where(rows < NE, e2, 0.0), axis=0, keepdims=True)

    @pl.when(i == 0)
    def _():
        sum_ref[...] = jnp.zeros_like(sum_ref)

    sum_ref[...] += part
    if last:
        oe[0][...] = _dot(de2, ow[...]) + ob[...]


def _make_edge(has_de, last):
    in_specs = [pl.BlockSpec((BE, DED), lambda i: (i, 0))]
    if has_de:
        in_specs.append(pl.BlockSpec((BE // 2, 2 * H), lambda i: (i, 0)))
    in_specs.append(pl.BlockSpec((BE, 2 * H), lambda i: (i, 0)))
    in_specs += [pl.BlockSpec((DED, H), lambda i: (0, 0)), _B64,
                 _W64, _W64, _B64, _W64, _B64, _W64, _B64,
                 pl.BlockSpec((H, DED), lambda i: (0, 0)),
                 pl.BlockSpec((1, DED), lambda i: (0, 0))]
    out_shape = [jax.ShapeDtypeStruct((EP // 2, 2 * H), jnp.float32),
                 jax.ShapeDtypeStruct((EP // 2, 2 * H), jnp.float32),
                 jax.ShapeDtypeStruct((1, H), jnp.float32)]
    out_specs = [pl.BlockSpec((BE // 2, 2 * H), lambda i: (i, 0)),
                 pl.BlockSpec((BE // 2, 2 * H), lambda i: (i, 0)),
                 pl.BlockSpec((1, H), lambda i: (0, 0))]
    if last:
        out_shape.append(jax.ShapeDtypeStruct((EP, DED), jnp.float32))
        out_specs.append(pl.BlockSpec((BE, DED), lambda i: (i, 0)))
    return pl.pallas_call(
        functools.partial(_edge_body, has_de, last),
        grid=(GE,), in_specs=in_specs, out_specs=out_specs, out_shape=out_shape)


_edge0 = _make_edge(False, False)
_edge1 = _make_edge(True, True)


def _node_body(has_dx, last, *refs):
    if has_dx:
        (lx_ref, dx_ref, agg_ref, wn1a, wn1b, wn1c, bn1, wn2, bn2, dw, db,
         wsa, wsb, wda, wdb, ow, ob, dx2_ref, xsd_ref, sum_ref, *ox) = refs
    else:
        (lx_ref, agg_ref, wn1a, wn1b, wn1c, bn1, wn2, bn2, dw, db,
         wsa, wsb, wda, wdb, ow, ob, dx2_ref, xsd_ref, sum_ref, *ox) = refs
        dx_ref = lx_ref
    i = pl.program_id(0)
    lx = lx_ref[...]
    agg = agg_ref[0] + agg_ref[1]
    h = _dot(lx, wn1a[...]) + _dot(dx_ref[...], wn1b[...]) + _dot(agg, wn1c[...]) + bn1[...]
    h = jnp.maximum(h, 0.0)
    h = jnp.maximum(_dot(h, wn2[...]) + bn2[...], 0.0)
    x2 = _ln(h)
    dx2 = jnp.maximum(_dot(x2, dw[...]) + db[...], 0.0)
    dx2_ref[...] = dx2
    xsd_ref[...] = jnp.concatenate(
        [_dot(lx, wsa[...]) + _dot(dx2, wsb[...]),
         _dot(lx, wda[...]) + _dot(dx2, wdb[...])], axis=1)
    part = jnp.sum(x2, axis=0, keepdims=True)

    @pl.when(i == 0)
    def _():
        sum_ref[...] = jnp.zeros_like(sum_ref)

    sum_ref[...] += part
    if last:
        ox[0][...] = _dot(dx2, ow[...]) + ob[...]


def _make_node(has_dx, last):
    in_specs = [pl.BlockSpec((BN, H), lambda i: (i, 0))]
    if has_dx:
        in_specs.append(pl.BlockSpec((BN, H), lambda i: (i, 0)))
    in_specs.append(pl.BlockSpec((2, BN, H), lambda i: (0, i, 0)))
    in_specs += [_W64, _W64, _W64, _B64, _W64, _B64, _W64, _B64,
                 _W64, _W64, _W64, _W64,
                 pl.BlockSpec((H, DXD), lambda i: (0, 0)),
                 pl.BlockSpec((1, DXD), lambda i: (0, 0))]
    out_shape = [jax.ShapeDtypeStruct((NN, H), jnp.float32),
                 jax.ShapeDtypeStruct((NN, 2 * H), jnp.float32),
                 jax.ShapeDtypeStruct((1, H), jnp.float32)]
    out_specs = [pl.BlockSpec((BN, H), lambda i: (i, 0)),
                 pl.BlockSpec((BN, 2 * H), lambda i: (i, 0)),
                 pl.BlockSpec((1, H), lambda i: (0, 0))]
    if last:
        out_shape.append(jax.ShapeDtypeStruct((NN, DXD), jnp.float32))
        out_specs.append(pl.BlockSpec((BN, DXD), lambda i: (i, 0)))
    return pl.pallas_call(
        functools.partial(_node_body, has_dx, last),
        grid=(GN,), in_specs=in_specs, out_specs=out_specs, out_shape=out_shape)


_node0 = _make_node(False, False)
_node1 = _make_node(True, True)


def _enc_e_body(e_ref, w_ref, b_ref, le_ref):
    le_ref[...] = jnp.maximum(_dot(e_ref[...], w_ref[...]) + b_ref[...], 0.0)


_enc_e = pl.pallas_call(
    _enc_e_body, grid=(GE,),
    in_specs=[pl.BlockSpec((BE, DED), lambda i: (i, 0)),
              pl.BlockSpec((DED, H), lambda i: (0, 0)),
              pl.BlockSpec((1, H), lambda i: (0, 0))],
    out_specs=pl.BlockSpec((BE, H), lambda i: (i, 0)),
    out_shape=jax.ShapeDtypeStruct((EP, H), jnp.float32))


def _enc_x_body(x_ref, w_ref, b_ref, ws_ref, wd_ref, lx_ref, xsd_ref):
    lx = jnp.maximum(_dot(x_ref[...], w_ref[...]) + b_ref[...], 0.0)
    lx_ref[...] = lx
    lxc = jnp.concatenate([lx, lx], axis=1)
    xsd_ref[...] = jnp.concatenate(
        [_dot(lxc, ws_ref[...]), _dot(lxc, wd_ref[...])], axis=1)


_enc_x = pl.pallas_call(
    _enc_x_body, grid=(GN,),
    in_specs=[pl.BlockSpec((BN, DXD), lambda i: (i, 0)),
              pl.BlockSpec((DXD, H), lambda i: (0, 0)),
              pl.BlockSpec((1, H), lambda i: (0, 0)),
              pl.BlockSpec((2 * H, H), lambda i: (0, 0)),
              pl.BlockSpec((2 * H, H), lambda i: (0, 0))],
    out_specs=[pl.BlockSpec((BN, H), lambda i: (i, 0)),
               pl.BlockSpec((BN, 2 * H), lambda i: (i, 0))],
    out_shape=[jax.ShapeDtypeStruct((NN, H), jnp.float32),
               jax.ShapeDtypeStruct((NN, 2 * H), jnp.float32)])


def _prep_g_body(g_ref, w_ref, b_ref, wge_ref, b1e_ref, wgn_ref, b1n_ref,
                 lg_ref, be_ref, bn_ref):
    lg = jnp.maximum(_dot(g_ref[...], w_ref[...]) + b_ref[...], 0.0)
    lg_ref[...] = lg
    lgc = jnp.concatenate([lg, lg], axis=1)
    be_ref[...] = _dot(lgc, wge_ref[...]) + b1e_ref[...]
    bn_ref[...] = _dot(lgc, wgn_ref[...]) + b1n_ref[...]


_prep_g = pl.pallas_call(
    _prep_g_body,
    out_shape=[jax.ShapeDtypeStruct((1, H), jnp.float32)] * 3)


def _glob_body(last, *refs):
    (lg_ref, dg_ref, se_ref, sx_ref, wg1, bg1, wg2, bg2, dw, db,
     wge, b1e, wgn, b1n, ow, ob, *outs) = refs
    gcat = jnp.concatenate([lg_ref[...], dg_ref[...]], axis=1)
    gin = jnp.concatenate([gcat, se_ref[...], sx_ref[...]], axis=1)
    h = jnp.maximum(_dot(gin, wg1[...]) + bg1[...], 0.0)
    h = jnp.maximum(_dot(h, wg2[...]) + bg2[...], 0.0)
    g2 = _ln(h)
    dg2 = jnp.maximum(_dot(g2, dw[...]) + db[...], 0.0)
    if last:
        outs[0][...] = _dot(dg2, ow[...]) + ob[...]
    else:
        dg_out, be_out, bn_out = outs
        dg_out[...] = dg2
        gcat2 = jnp.concatenate([lg_ref[...], dg2], axis=1)
        be_out[...] = _dot(gcat2, wge[...]) + b1e[...]
        bn_out[...] = _dot(gcat2, wgn[...]) + b1n[...]


_glob0 = pl.pallas_call(
    functools.partial(_glob_body, False),
    out_shape=[jax.ShapeDtypeStruct((1, H), jnp.float32)] * 3)
_glob1 = pl.pallas_call(
    functools.partial(_glob_body, True),
    out_shape=[jax.ShapeDtypeStruct((1, DGD), jnp.float32)])


def _row(b):
    return b[None, :]


def kernel(x, e, g, params, edges, node_idx, edge_idx, steps):
    p = params
    src = edges[0]
    dst = edges[1]
    padn = EP - NE
    zpad = jnp.zeros((padn,), jnp.int32)
    idx_gather = jnp.stack(
        [2 * jnp.concatenate([src, zpad]),
         2 * jnp.concatenate([dst, zpad]) + 1], axis=1).reshape(-1)
    idx_scatter = jnp.concatenate(
        [dst, jnp.full((padn,), NN, jnp.int32)]).reshape(NTILES, SCH, CH)
    zeros_np = jnp.zeros((NP, H), jnp.float32)
    e_pad = jnp.pad(e, ((0, padn), (0, 0)))
    r = jnp.arange(EP, dtype=jnp.int32)
    perm = (r // BE) * BE + (r % 2) * (BE // 2) + (r % BE) // 2
    idx_scatter = jnp.take(idx_scatter.reshape(-1), perm).reshape(
        NTILES, SCH, CH)

    w1 = p['core_e_W1']
    wn1 = p['core_n_W1']

    lx, xsd = _enc_x(x, p['enc_x_W'], _row(p['enc_x_b']),
                     w1[128:256], w1[256:384])
    lg, be_b, bn_b = _prep_g(g, p['enc_g_W'], _row(p['enc_g_b']),
                             w1[384:512], _row(p['core_e_b1']),
                             wn1[192:320], _row(p['core_n_b1']))

    enc_e = (p['enc_e_W'], _row(p['enc_e_b']))
    edge_w = (w1[0:64], w1[64:128])
    edge_tail = (p['core_e_W2'], _row(p['core_e_b2']),
                 p['dec_e_W'], _row(p['dec_e_b']),
                 p['out_e_W'], _row(p['out_e_b']))
    node_w = (wn1[0:64], wn1[64:128], wn1[128:192])
    node_tail = (p['core_n_W2'], _row(p['core_n_b2']),
                 p['dec_x_W'], _row(p['dec_x_b']),
                 w1[128:192], w1[192:256], w1[256:320], w1[320:384],
                 p['out_x_W'], _row(p['out_x_b']))
    glob_w = (p['core_g_W1'], _row(p['core_g_b1']),
              p['core_g_W2'], _row(p['core_g_b2']),
              p['dec_g_W'], _row(p['dec_g_b']),
              w1[384:512], _row(p['core_e_b1']),
              wn1[192:320], _row(p['core_n_b1']),
              p['out_g_W'], _row(p['out_g_b']))

    gsd = _sc_gather(xsd.reshape(2 * NN, H), idx_gather).reshape(EP, 2 * H)
    e2, de, se = _edge0(e_pad, gsd, *enc_e, edge_w[0],
                        edge_w[1], be_b, *edge_tail)
    agg = _sc_scatter(e2.reshape(EP, H), idx_scatter, zeros_np)
    dx, xsd, sx = _node0(lx, agg, *node_w, bn_b, *node_tail)
    dg, be_b, bn_b = _glob0(lg, lg, se, sx, *glob_w)

    gsd = _sc_gather(xsd.reshape(2 * NN, H), idx_gather).reshape(EP, 2 * H)
    e2, de, se, oe = _edge1(e_pad, de, gsd, *enc_e, edge_w[0],
                            edge_w[1], be_b, *edge_tail)
    agg = _sc_scatter(e2.reshape(EP, H), idx_scatter, zeros_np)
    dx, _, sx, ox = _node1(lx, dx, agg, *node_w, bn_b, *node_tail)
    (og,) = _glob1(lg, dg, se, sx, *glob_w)

    return (oe[:NE], ox, og)


# speedup vs baseline: 2.1965x; 1.1484x over previous
import functools

import jax
import jax.numpy as jnp
from jax import lax
from jax.experimental import pallas as pl
from jax.experimental.pallas import tpu as pltpu
from jax.experimental.pallas import tpu_sc as plsc

NN = 10000
NE = 160000
DXD = 128
DED = 16
DGD = 16
H = 64

NTILES = 32
CH = 128
EP = 163840
SCH = EP // NTILES // CH
GCH = 2 * EP // NTILES // CH
NP = 10016
ZR = NP // 16
OR_ = NN // 16

BE = 2048
GE = EP // BE
BN = 2000
GN = NN // BN


@functools.cache
def _sc_kernels():
    mesh = plsc.VectorSubcoreMesh(core_axis_name="c", subcore_axis_name="s")

    nbg = 2
    EPT = EP // NTILES
    NCH = EPT // CH

    @functools.partial(
        pl.kernel,
        out_type=jax.ShapeDtypeStruct((EP, 2 * H), jnp.float32),
        mesh=mesh,
        compiler_params=pltpu.CompilerParams(use_tc_tiling_on_sc=False),
        scratch_types=[
            pltpu.VMEM_SHARED((2 * NN, H), jnp.float32),
            pltpu.VMEM((2, EPT), jnp.int32),
            pltpu.VMEM((nbg, 2, CH, H), jnp.float32),
        ] + [pltpu.SemaphoreType.DMA] * (2 * nbg),
    )
    def sc_gather(table, edges_p, out, shared_t, idx_v, rows_v, *sems):
        gsems, wsems = sems[:nbg], sems[nbg:]
        cid = lax.axis_index("c")
        sid = lax.axis_index("s")
        wid = sid * 2 + cid
        base = wid * EPT
        trows = 2 * NN // 16
        pltpu.sync_copy(table.at[pl.ds(sid * trows, trows)],
                        shared_t.at[pl.ds(sid * trows, trows)])
        pltpu.sync_copy(edges_p.at[0, pl.ds(base, EPT)], idx_v.at[0])
        pltpu.sync_copy(edges_p.at[1, pl.ds(base, EPT)], idx_v.at[1])

        def xform(k, carry):
            s = idx_v[0, pl.ds(k * 16, 16)]
            idx_v[0, pl.ds(k * 16, 16)] = 2 * s
            d = idx_v[1, pl.ds(k * 16, 16)]
            idx_v[1, pl.ds(k * 16, 16)] = 2 * d + 1
            return carry

        lax.fori_loop(0, EPT // 16, xform, 0)
        plsc.subcore_barrier()

        def gather_src(j, half):
            return shared_t.at[idx_v.at[half, pl.ds(j * CH, CH)]]

        def out_dst(j, half):
            return out.at[pl.ds(base + j * CH, CH), pl.ds(half * H, H)]

        def body(g, carry):
            for b in range(nbg):
                j = g * nbg + b

                @pl.when(g > 0)
                def _():
                    for half in range(2):
                        pltpu.make_async_copy(
                            rows_v.at[b, half], out_dst(j - nbg, half),
                            wsems[b]).wait()

                for half in range(2):
                    pltpu.async_copy(gather_src(j, half), rows_v.at[b, half],
                                     gsems[b])
            for b in range(nbg):
                j = g * nbg + b
                for half in range(2):
                    pltpu.make_async_copy(gather_src(j, half),
                                          rows_v.at[b, half], gsems[b]).wait()
                    pltpu.async_copy(rows_v.at[b, half], out_dst(j, half),
                                     wsems[b])
            return carry

        lax.fori_loop(0, NCH // nbg, body, 0)
        for b in range(nbg):
            j = (NCH // nbg - 1) * nbg + b
            for half in range(2):
                pltpu.make_async_copy(rows_v.at[b, half], out_dst(j, half),
                                      wsems[b]).wait()

    nbs = 4

    @functools.partial(
        pl.kernel,
        out_type=jax.ShapeDtypeStruct((2, NN, H), jnp.float32),
        mesh=mesh,
        compiler_params=pltpu.CompilerParams(use_tc_tiling_on_sc=False),
        scratch_types=[
            pltpu.VMEM_SHARED((NP, H), jnp.float32),
            pltpu.VMEM((SCH, CH), jnp.int32),
            pltpu.VMEM((nbs, CH, H), jnp.float32),
        ] + [pltpu.SemaphoreType.DMA] * (2 * nbs),
    )
    def sc_scatter(e2, idx3, zeros_hbm, out, shared, idx_v, rows_v, *sems):
        rsems, ssems = sems[:nbs], sems[nbs:]
        cid = lax.axis_index("c")
        sid = lax.axis_index("s")
        wid = sid * 2 + cid
        pltpu.sync_copy(zeros_hbm.at[pl.ds(sid * ZR, ZR)],
                        shared.at[pl.ds(sid * ZR, ZR)])
        pltpu.sync_copy(idx3.at[wid], idx_v)
        plsc.subcore_barrier()
        base = wid * (EP // NTILES)

        def body(g, carry):
            for b in range(nbs):
                j = g * nbs + b

                @pl.when(g > 0)
                def _():
                    pltpu.make_async_copy(
                        rows_v.at[b], shared.at[idx_v.at[j - nbs]], ssems[b]).wait()

                pltpu.async_copy(e2.at[pl.ds(base + j * CH, CH)],
                                 rows_v.at[b], rsems[b])
            for b in range(nbs):
                j = g * nbs + b
                pltpu.make_async_copy(e2.at[pl.ds(base + j * CH, CH)],
                                      rows_v.at[b], rsems[b]).wait()
                pltpu.async_copy(rows_v.at[b], shared.at[idx_v.at[j]],
                                 ssems[b], add=True)
            return carry

        ngroups = SCH // nbs
        lax.fori_loop(0, ngroups, body, 0)
        for b in range(nbs):
            j = (ngroups - 1) * nbs + b
            pltpu.make_async_copy(rows_v.at[b], shared.at[idx_v.at[j]],
                                  ssems[b]).wait()
        plsc.subcore_barrier()
        pltpu.sync_copy(shared.at[pl.ds(sid * OR_, OR_)],
                        out.at[cid, pl.ds(sid * OR_, OR_)])

    return sc_gather, sc_scatter


def _sc_gather(table, idx):
    return _sc_kernels()[0](table, idx)


def _sc_scatter(e2, idx3, zeros_np):
    return _sc_kernels()[1](e2, idx3, zeros_np)


def _ln(h):
    m = jnp.mean(h, axis=-1, keepdims=True)
    v = jnp.var(h, axis=-1, keepdims=True)
    return (h - m) / jnp.sqrt(v + 1e-5)


def _dot(a, b):
    return jax.lax.dot_general(a, b, (((1,), (0,)), ((), ())),
                               preferred_element_type=jnp.float32)


_W64 = pl.BlockSpec((H, H), lambda i: (0, 0))
_B64 = pl.BlockSpec((1, H), lambda i: (0, 0))


def _edge_body(has_de, last, *refs):
    if has_de:
        (e_ref, de_ref, gsd_ref, we, be, w1ab, b1, w2, b2, dw, db, ow, ob,
         e2_ref, de2_ref, sum_ref, *oe) = refs
    else:
        (e_ref, gsd_ref, we, be, w1ab, b1, w2, b2, dw, db, ow, ob,
         e2_ref, de2_ref, sum_ref, *oe) = refs
        de_ref = None
    i = pl.program_id(0)
    le = jnp.maximum(_dot(e_ref[...], we[...]) + be[...], 0.0)
    if has_de:
        dp = de_ref[...]
        de = jnp.concatenate([dp[:, :H], dp[:, H:]], axis=0)
    else:
        de = le
    g = gsd_ref[...]
    h = _dot(jnp.concatenate([le, de], axis=1), w1ab[...])
    h = h + g[:, :H] + g[:, H:] + b1[...]
    h = jnp.maximum(h, 0.0)
    h = jnp.maximum(_dot(h, w2[...]) + b2[...], 0.0)
    e2 = _ln(h)
    e2_ref[...] = jnp.concatenate([e2[:BE // 2], e2[BE // 2:]], axis=1)
    de2 = jnp.maximum(_dot(e2, dw[...]) + db[...], 0.0)
    de2_ref[...] = jnp.concatenate([de2[:BE // 2], de2[BE // 2:]], axis=1)
    rows = i * BE + lax.broadcasted_iota(jnp.int32, (BE, 1), 0)
    part = jnp.sum(jnp.where(rows < NE, e2, 0.0), axis=0, keepdims=True)

    @pl.when(i == 0)
    def _():
        sum_ref[...] = jnp.zeros_like(sum_ref)

    sum_ref[...] += part
    if last:
        oe[0][...] = _dot(de2, ow[...]) + ob[...]


def _make_edge(has_de, last):
    in_specs = [pl.BlockSpec((BE, DED), lambda i: (i, 0))]
    if has_de:
        in_specs.append(pl.BlockSpec((BE // 2, 2 * H), lambda i: (i, 0)))
    in_specs.append(pl.BlockSpec((BE, 2 * H), lambda i: (i, 0)))
    in_specs += [pl.BlockSpec((DED, H), lambda i: (0, 0)), _B64,
                 pl.BlockSpec((2 * H, H), lambda i: (0, 0)),
                 _B64, _W64, _B64, _W64, _B64,
                 pl.BlockSpec((H, DED), lambda i: (0, 0)),
                 pl.BlockSpec((1, DED), lambda i: (0, 0))]
    out_shape = [jax.ShapeDtypeStruct((EP // 2, 2 * H), jnp.float32),
                 jax.ShapeDtypeStruct((EP // 2, 2 * H), jnp.float32),
                 jax.ShapeDtypeStruct((1, H), jnp.float32)]
    out_specs = [pl.BlockSpec((BE // 2, 2 * H), lambda i: (i, 0)),
                 pl.BlockSpec((BE // 2, 2 * H), lambda i: (i, 0)),
                 pl.BlockSpec((1, H), lambda i: (0, 0))]
    if last:
        out_shape.append(jax.ShapeDtypeStruct((EP, DED), jnp.float32))
        out_specs.append(pl.BlockSpec((BE, DED), lambda i: (i, 0)))
    return pl.pallas_call(
        functools.partial(_edge_body, has_de, last),
        grid=(GE,), in_specs=in_specs, out_specs=out_specs, out_shape=out_shape)


_edge0 = _make_edge(False, False)
_edge1 = _make_edge(True, True)


def _node_body(has_dx, last, *refs):
    if has_dx:
        (lx_ref, dx_ref, agg_ref, wn1a, wn1b, wn1c, bn1, wn2, bn2, dw, db,
         wsa, wsb, wda, wdb, ow, ob, dx2_ref, xsd_ref, sum_ref, *ox) = refs
    else:
        (lx_ref, agg_ref, wn1a, wn1b, wn1c, bn1, wn2, bn2, dw, db,
         wsa, wsb, wda, wdb, ow, ob, dx2_ref, xsd_ref, sum_ref, *ox) = refs
        dx_ref = lx_ref
    i = pl.program_id(0)
    lx = lx_ref[...]
    agg = agg_ref[0] + agg_ref[1]
    h = _dot(lx, wn1a[...]) + _dot(dx_ref[...], wn1b[...]) + _dot(agg, wn1c[...]) + bn1[...]
    h = jnp.maximum(h, 0.0)
    h = jnp.maximum(_dot(h, wn2[...]) + bn2[...], 0.0)
    x2 = _ln(h)
    dx2 = jnp.maximum(_dot(x2, dw[...]) + db[...], 0.0)
    dx2_ref[...] = dx2
    xsd_ref[...] = jnp.concatenate(
        [_dot(lx, wsa[...]) + _dot(dx2, wsb[...]),
         _dot(lx, wda[...]) + _dot(dx2, wdb[...])], axis=1)
    part = jnp.sum(x2, axis=0, keepdims=True)

    @pl.when(i == 0)
    def _():
        sum_ref[...] = jnp.zeros_like(sum_ref)

    sum_ref[...] += part
    if last:
        ox[0][...] = _dot(dx2, ow[...]) + ob[...]


def _make_node(has_dx, last):
    in_specs = [pl.BlockSpec((BN, H), lambda i: (i, 0))]
    if has_dx:
        in_specs.append(pl.BlockSpec((BN, H), lambda i: (i, 0)))
    in_specs.append(pl.BlockSpec((2, BN, H), lambda i: (0, i, 0)))
    in_specs += [_W64, _W64, _W64, _B64, _W64, _B64, _W64, _B64,
                 _W64, _W64, _W64, _W64,
                 pl.BlockSpec((H, DXD), lambda i: (0, 0)),
                 pl.BlockSpec((1, DXD), lambda i: (0, 0))]
    out_shape = [jax.ShapeDtypeStruct((NN, H), jnp.float32),
                 jax.ShapeDtypeStruct((NN, 2 * H), jnp.float32),
                 jax.ShapeDtypeStruct((1, H), jnp.float32)]
    out_specs = [pl.BlockSpec((BN, H), lambda i: (i, 0)),
                 pl.BlockSpec((BN, 2 * H), lambda i: (i, 0)),
                 pl.BlockSpec((1, H), lambda i: (0, 0))]
    if last:
        out_shape.append(jax.ShapeDtypeStruct((NN, DXD), jnp.float32))
        out_specs.append(pl.BlockSpec((BN, DXD), lambda i: (i, 0)))
    return pl.pallas_call(
        functools.partial(_node_body, has_dx, last),
        grid=(GN,), in_specs=in_specs, out_specs=out_specs, out_shape=out_shape)


_node0 = _make_node(False, False)
_node1 = _make_node(True, True)


def _enc_e_body(e_ref, w_ref, b_ref, le_ref):
    le_ref[...] = jnp.maximum(_dot(e_ref[...], w_ref[...]) + b_ref[...], 0.0)


_enc_e = pl.pallas_call(
    _enc_e_body, grid=(GE,),
    in_specs=[pl.BlockSpec((BE, DED), lambda i: (i, 0)),
              pl.BlockSpec((DED, H), lambda i: (0, 0)),
              pl.BlockSpec((1, H), lambda i: (0, 0))],
    out_specs=pl.BlockSpec((BE, H), lambda i: (i, 0)),
    out_shape=jax.ShapeDtypeStruct((EP, H), jnp.float32))


def _enc_x_body(x_ref, w_ref, b_ref, ws_ref, wd_ref, lx_ref, xsd_ref):
    lx = jnp.maximum(_dot(x_ref[...], w_ref[...]) + b_ref[...], 0.0)
    lx_ref[...] = lx
    lxc = jnp.concatenate([lx, lx], axis=1)
    xsd_ref[...] = jnp.concatenate(
        [_dot(lxc, ws_ref[...]), _dot(lxc, wd_ref[...])], axis=1)


_enc_x = pl.pallas_call(
    _enc_x_body, grid=(GN,),
    in_specs=[pl.BlockSpec((BN, DXD), lambda i: (i, 0)),
              pl.BlockSpec((DXD, H), lambda i: (0, 0)),
              pl.BlockSpec((1, H), lambda i: (0, 0)),
              pl.BlockSpec((2 * H, H), lambda i: (0, 0)),
              pl.BlockSpec((2 * H, H), lambda i: (0, 0))],
    out_specs=[pl.BlockSpec((BN, H), lambda i: (i, 0)),
               pl.BlockSpec((BN, 2 * H), lambda i: (i, 0))],
    out_shape=[jax.ShapeDtypeStruct((NN, H), jnp.float32),
               jax.ShapeDtypeStruct((NN, 2 * H), jnp.float32)])


def _prep_g_body(g_ref, w_ref, b_ref, wge_ref, b1e_ref, wgn_ref, b1n_ref,
                 lg_ref, be_ref, bn_ref):
    lg = jnp.maximum(_dot(g_ref[...], w_ref[...]) + b_ref[...], 0.0)
    lg_ref[...] = lg
    lgc = jnp.concatenate([lg, lg], axis=1)
    be_ref[...] = _dot(lgc, wge_ref[...]) + b1e_ref[...]
    bn_ref[...] = _dot(lgc, wgn_ref[...]) + b1n_ref[...]


_prep_g = pl.pallas_call(
    _prep_g_body,
    out_shape=[jax.ShapeDtypeStruct((1, H), jnp.float32)] * 3)


def _glob_body(last, *refs):
    (lg_ref, dg_ref, se_ref, sx_ref, wg1, bg1, wg2, bg2, dw, db,
     wge, b1e, wgn, b1n, ow, ob, *outs) = refs
    gcat = jnp.concatenate([lg_ref[...], dg_ref[...]], axis=1)
    gin = jnp.concatenate([gcat, se_ref[...], sx_ref[...]], axis=1)
    h = jnp.maximum(_dot(gin, wg1[...]) + bg1[...], 0.0)
    h = jnp.maximum(_dot(h, wg2[...]) + bg2[...], 0.0)
    g2 = _ln(h)
    dg2 = jnp.maximum(_dot(g2, dw[...]) + db[...], 0.0)
    if last:
        outs[0][...] = _dot(dg2, ow[...]) + ob[...]
    else:
        dg_out, be_out, bn_out = outs
        dg_out[...] = dg2
        gcat2 = jnp.concatenate([lg_ref[...], dg2], axis=1)
        be_out[...] = _dot(gcat2, wge[...]) + b1e[...]
        bn_out[...] = _dot(gcat2, wgn[...]) + b1n[...]


_glob0 = pl.pallas_call(
    functools.partial(_glob_body, False),
    out_shape=[jax.ShapeDtypeStruct((1, H), jnp.float32)] * 3)
_glob1 = pl.pallas_call(
    functools.partial(_glob_body, True),
    out_shape=[jax.ShapeDtypeStruct((1, DGD), jnp.float32)])


def _row(b):
    return b[None, :]


def kernel(x, e, g, params, edges, node_idx, edge_idx, steps):
    p = params
    src = edges[0]
    dst = edges[1]
    padn = EP - NE
    zpad = jnp.zeros((padn,), jnp.int32)
    edges_p = jnp.concatenate([edges, jnp.zeros((2, padn), jnp.int32)], axis=1)
    idx_scatter = jnp.concatenate(
        [dst, jnp.full((padn,), NN, jnp.int32)]).reshape(NTILES, SCH, CH)
    zeros_np = jnp.zeros((NP, H), jnp.float32)
    e_pad = jnp.pad(e, ((0, padn), (0, 0)))
    r = jnp.arange(EP, dtype=jnp.int32)
    perm = (r // BE) * BE + (r % 2) * (BE // 2) + (r % BE) // 2
    idx_scatter = jnp.take(idx_scatter.reshape(-1), perm).reshape(
        NTILES, SCH, CH)

    w1 = p['core_e_W1']
    wn1 = p['core_n_W1']

    lx, xsd = _enc_x(x, p['enc_x_W'], _row(p['enc_x_b']),
                     w1[128:256], w1[256:384])
    lg, be_b, bn_b = _prep_g(g, p['enc_g_W'], _row(p['enc_g_b']),
                             w1[384:512], _row(p['core_e_b1']),
                             wn1[192:320], _row(p['core_n_b1']))

    enc_e = (p['enc_e_W'], _row(p['enc_e_b']))
    edge_w = w1[0:128]
    edge_tail = (p['core_e_W2'], _row(p['core_e_b2']),
                 p['dec_e_W'], _row(p['dec_e_b']),
                 p['out_e_W'], _row(p['out_e_b']))
    node_w = (wn1[0:64], wn1[64:128], wn1[128:192])
    node_tail = (p['core_n_W2'], _row(p['core_n_b2']),
                 p['dec_x_W'], _row(p['dec_x_b']),
                 w1[128:192], w1[192:256], w1[256:320], w1[320:384],
                 p['out_x_W'], _row(p['out_x_b']))
    glob_w = (p['core_g_W1'], _row(p['core_g_b1']),
              p['core_g_W2'], _row(p['core_g_b2']),
              p['dec_g_W'], _row(p['dec_g_b']),
              w1[384:512], _row(p['core_e_b1']),
              wn1[192:320], _row(p['core_n_b1']),
              p['out_g_W'], _row(p['out_g_b']))

    gsd = _sc_gather(xsd.reshape(2 * NN, H), edges_p)
    e2, de, se = _edge0(e_pad, gsd, *enc_e, edge_w, be_b, *edge_tail)
    agg = _sc_scatter(e2.reshape(EP, H), idx_scatter, zeros_np)
    dx, xsd, sx = _node0(lx, agg, *node_w, bn_b, *node_tail)
    dg, be_b, bn_b = _glob0(lg, lg, se, sx, *glob_w)

    gsd = _sc_gather(xsd.reshape(2 * NN, H), edges_p)
    e2, de, se, oe = _edge1(e_pad, de, gsd, *enc_e, edge_w, be_b, *edge_tail)
    agg = _sc_scatter(e2.reshape(EP, H), idx_scatter, zeros_np)
    dx, _, sx, ox = _node1(lx, dx, agg, *node_w, bn_b, *node_tail)
    (og,) = _glob1(lg, dg, se, sx, *glob_w)

    return (oe[:NE], ox, og)
